# Initial kernel scaffold; baseline (speedup 1.0000x reference)
import jax
import jax.numpy as jnp
from jax.experimental import pallas as pl


def kernel(question, response, mask, q_neighbors, s_neighbors, qs_table, emb_q, emb_s, emb_r, ft_W, ft_b, agg_W, agg_b, last_W, last_b, Wih, Whh, bih, bhh, q_W, q_b, k_W, k_b, w_W, w_b):
    raise NotImplementedError("write your pallas kernel here")


# speedup vs baseline: 8.8741x; 8.8741x over previous
import functools

import jax
import jax.numpy as jnp
import numpy as np
from jax import lax
from jax.experimental import pallas as pl
from jax.experimental.pallas import tpu as pltpu
from jax.experimental.pallas import tpu_sc as plsc

NUM_Q = 20000
NUM_S = 500
EMB = 128
B = 1024
S = 20
RANK_K = 10
T = S - 1
BQ = 512
NSP = 512
GRID_Q = (NUM_Q + BQ - 1) // BQ
NC, NS_SC = 2, 16
NW = NC * NS_SC
HI = lax.Precision.HIGHEST
F32 = jnp.float32


def _dot(a, b):
    return jnp.dot(a, b, preferred_element_type=F32, precision=HI)


def _sc_gather(table, idx, rows, chunk):
    per_w = rows // NW
    n_chunks = per_w // chunk
    mesh = plsc.VectorSubcoreMesh(core_axis_name="c", subcore_axis_name="s")

    @functools.partial(
        pl.kernel,
        out_type=jax.ShapeDtypeStruct((rows, EMB), F32),
        mesh=mesh,
        scratch_types=[
            pltpu.VMEM((chunk,), jnp.int32),
            pltpu.VMEM((chunk, EMB), F32),
            pltpu.SemaphoreType.DMA,
        ],
    )
    def gather(table_hbm, idx_hbm, out_hbm, idx_v, rows_v, sem):
        wid = lax.axis_index("s") * NC + lax.axis_index("c")
        base = wid * per_w
        for k in range(n_chunks):
            off = base + k * chunk
            pltpu.sync_copy(idx_hbm.at[pl.ds(off, chunk)], idx_v)
            pltpu.async_copy(table_hbm.at[idx_v], rows_v, sem).wait()
            pltpu.sync_copy(rows_v, out_hbm.at[pl.ds(off, chunk)])

    return gather(table, idx)


def _passA_body(qn_ref, eq_ref, es_ref, W2_ref, b2_ref, W0_ref, b0_ref,
                tall_ref, t0a_ref):
    qn = qn_ref[...]
    iot = lax.broadcasted_iota(jnp.int32, (BQ, NSP), 1)
    counts = jnp.zeros((BQ, NSP), F32)
    for j in range(4):
        counts += (qn[:, j:j + 1] == iot).astype(F32)
    qmean = _dot(counts, es_ref[...]) * 0.25
    x = eq_ref[...] + qmean
    tall_ref[0] = eq_ref[...]
    tall_ref[1] = jnp.tanh(_dot(x, W2_ref[...]) + b2_ref[...])
    t0a_ref[...] = jnp.tanh(_dot(x, W0_ref[...]) + b0_ref[...])


def _passA(qn, emb_q, es512, W2, b2, W0, b0):
    return pl.pallas_call(
        _passA_body,
        grid=(GRID_Q,),
        in_specs=[
            pl.BlockSpec((BQ, 4), lambda i: (i, 0)),
            pl.BlockSpec((BQ, EMB), lambda i: (i, 0)),
            pl.BlockSpec((NSP, EMB), lambda i: (0, 0)),
            pl.BlockSpec((EMB, EMB), lambda i: (0, 0)),
            pl.BlockSpec((1, EMB), lambda i: (0, 0)),
            pl.BlockSpec((EMB, EMB), lambda i: (0, 0)),
            pl.BlockSpec((1, EMB), lambda i: (0, 0)),
        ],
        out_specs=[
            pl.BlockSpec((2, BQ, EMB), lambda i: (0, i, 0)),
            pl.BlockSpec((BQ, EMB), lambda i: (i, 0)),
        ],
        out_shape=[
            jax.ShapeDtypeStruct((2, NUM_Q, EMB), F32),
            jax.ShapeDtypeStruct((NUM_Q, EMB), F32),
        ],
    )(qn, emb_q, es512, W2, b2, W0, b0)


def _passB_body(es_ref, g_ref, W1_ref, b1_ref, t1a_ref, t1b_ref):
    g = g_ref[...]
    sm0 = (g[0] + g[1] + g[2] + g[3]) * 0.25
    t1a = jnp.tanh(_dot(es_ref[...] + sm0, W1_ref[...]) + b1_ref[...])
    sm1 = (g[4] + g[5] + g[6] + g[7]) * 0.25
    t1b = jnp.tanh(_dot(t1a + sm1, W1_ref[...]) + b1_ref[...])
    t1a_ref[...] = t1a
    t1b_ref[...] = t1b


def _passB(es512, g, W1, b1):
    return pl.pallas_call(
        _passB_body,
        out_shape=[
            jax.ShapeDtypeStruct((NSP, EMB), F32),
            jax.ShapeDtypeStruct((NSP, EMB), F32),
        ],
    )(es512, g, W1, b1)


def _passC_body(qn_ref, eq_ref, t0a_ref, qs_ref, t1a_ref, t1b_ref, es_ref,
                W0_ref, b0_ref, lw_ref, lb_ref, fw_ref, fb_ref, tall_ref):
    qn = qn_ref[...]
    iot = lax.broadcasted_iota(jnp.int32, (BQ, NSP), 1)
    counts = jnp.zeros((BQ, NSP), F32)
    for j in range(4):
        counts += (qn[:, j:j + 1] == iot).astype(F32)
    qm1 = _dot(counts, t1a_ref[...]) * 0.25
    t0b = jnp.tanh(_dot(t0a_ref[...] + qm1, W0_ref[...]) + b0_ref[...])
    qm2 = _dot(counts, t1b_ref[...]) * 0.25
    t0c = jnp.tanh(_dot(t0b + qm2, W0_ref[...]) + b0_ref[...])
    qfin = jnp.tanh(_dot(t0c, lw_ref[...]) + lb_ref[...])
    tall_ref[0] = jnp.maximum(_dot(eq_ref[...], fw_ref[...]) + fb_ref[...], 0.0)
    tall_ref[1] = jnp.maximum(_dot(qfin, fw_ref[...]) + fb_ref[...], 0.0)
    tall_ref[2] = eq_ref[...]
    qs = qs_ref[...]
    esum = _dot(qs, es_ref[...])
    rs = jnp.sum(qs, axis=1, keepdims=True)
    tall_ref[3] = esum / jnp.maximum(rs, 1.0)


def _passC(qn, emb_q, t0a, qs_table, t1a, t1b, emb_s, W0, b0, lw, lb, fw, fb):
    return pl.pallas_call(
        _passC_body,
        grid=(GRID_Q,),
        in_specs=[
            pl.BlockSpec((BQ, 4), lambda i: (i, 0)),
            pl.BlockSpec((BQ, EMB), lambda i: (i, 0)),
            pl.BlockSpec((BQ, EMB), lambda i: (i, 0)),
            pl.BlockSpec((BQ, NUM_S), lambda i: (i, 0)),
            pl.BlockSpec((NSP, EMB), lambda i: (0, 0)),
            pl.BlockSpec((NSP, EMB), lambda i: (0, 0)),
            pl.BlockSpec((NUM_S, EMB), lambda i: (0, 0)),
            pl.BlockSpec((EMB, EMB), lambda i: (0, 0)),
            pl.BlockSpec((1, EMB), lambda i: (0, 0)),
            pl.BlockSpec((EMB, EMB), lambda i: (0, 0)),
            pl.BlockSpec((1, EMB), lambda i: (0, 0)),
            pl.BlockSpec((EMB, EMB), lambda i: (0, 0)),
            pl.BlockSpec((1, EMB), lambda i: (0, 0)),
        ],
        out_specs=pl.BlockSpec((4, BQ, EMB), lambda i: (0, i, 0)),
        out_shape=jax.ShapeDtypeStruct((4, NUM_Q, EMB), F32),
    )(qn, emb_q, t0a, qs_table, t1a, t1b, emb_s, W0, b0, lw, lb, fw, fb)


def _passR_body(xsel_ref, eqn_ref, esk_ref, resp_ref, WihA_ref, WihB_ref,
                Whh_ref, bsum_ref, qW_ref, qb_ref, kW_ref, kb_ref, wv_ref,
                er_ref, h0_ref, c0_ref, out_ref, h_s, c_s, qring, lqring):
    t = pl.program_id(0)

    @pl.when(t == 0)
    def _init():
        h_s[...] = h0_ref[...]
        c_s[...] = c0_ref[...]

    rW = _dot(er_ref[...], WihB_ref[...])
    resp = resp_ref[0]
    gates = (_dot(xsel_ref[0], WihA_ref[...]) + _dot(h_s[...], Whh_ref[...])
             + bsum_ref[...] + rW[0:1] + resp * (rW[1:2] - rW[0:1]))
    ig = jax.nn.sigmoid(gates[:, 0:EMB])
    fg = jax.nn.sigmoid(gates[:, EMB:2 * EMB])
    gg = jnp.tanh(gates[:, 2 * EMB:3 * EMB])
    og = jax.nn.sigmoid(gates[:, 3 * EMB:4 * EMB])
    c = fg * c_s[...] + ig * gg
    h = og * jnp.tanh(c)
    c_s[...] = c
    h_s[...] = h

    w1 = wv_ref[0:1]
    w2 = wv_ref[1:2]
    qh = _dot(h, qW_ref[...]) + qb_ref[...]
    lq_t = jnp.sum(qh * w1, axis=1)
    slot = lax.rem(t, RANK_K)
    for s_i in range(RANK_K):
        @pl.when(slot == s_i)
        def _store(s_i=s_i):
            qring[s_i] = qh
            lqring[s_i] = lq_t

    Km0 = _dot(eqn_ref[0], kW_ref[...]) + kb_ref[...]
    Km1 = _dot(esk_ref[0], kW_ref[...]) + kb_ref[...]
    lk0 = jnp.sum(Km0 * w2, axis=1)
    lk1 = jnp.sum(Km1 * w2, axis=1)

    ls, gs = [], []
    mx = jnp.full((B,), -1e30, F32)
    for s_i in range(RANK_K):
        valid = jnp.logical_or(s_i <= t, t >= RANK_K)
        qrow = qring[s_i]
        lq_s = lqring[s_i]
        for km, lk in ((Km0, lk0), (Km1, lk1)):
            l = jnp.where(valid, lq_s + lk, -1e30)
            g = jnp.where(valid,
                          jax.nn.sigmoid(jnp.sum(qrow * km, axis=1)), 0.0)
            ls.append(l)
            gs.append(g)
            mx = jnp.maximum(mx, l)
    num = jnp.zeros((B,), F32)
    den = jnp.zeros((B,), F32)
    for l, g in zip(ls, gs):
        e = jnp.exp(l - mx)
        num += e * g
        den += e
    out_ref[0, 0] = num / den


def _passR(xsel, eqn, esk, respf, WihA, WihB, Whh, bsum, qW, qb, kW, kb,
           wv, emb_r, h0, c0):
    full = lambda shape: pl.BlockSpec(shape, lambda t: tuple(0 for _ in shape))
    return pl.pallas_call(
        _passR_body,
        grid=(T,),
        in_specs=[
            pl.BlockSpec((1, B, EMB), lambda t: (t, 0, 0)),
            pl.BlockSpec((1, B, EMB), lambda t: (t, 0, 0)),
            pl.BlockSpec((1, B, EMB), lambda t: (t, 0, 0)),
            pl.BlockSpec((1, B, 1), lambda t: (t, 0, 0)),
            full((EMB, 4 * EMB)),
            full((EMB, 4 * EMB)),
            full((EMB, 4 * EMB)),
            full((1, 4 * EMB)),
            full((EMB, EMB)),
            full((1, EMB)),
            full((EMB, EMB)),
            full((1, EMB)),
            full((2, EMB)),
            full((2, EMB)),
            full((B, EMB)),
            full((B, EMB)),
        ],
        out_specs=pl.BlockSpec((1, 1, B), lambda t: (t, 0, 0)),
        out_shape=jax.ShapeDtypeStruct((T, 1, B), F32),
        scratch_shapes=[
            pltpu.VMEM((B, EMB), F32),
            pltpu.VMEM((B, EMB), F32),
            pltpu.VMEM((RANK_K, B, EMB), F32),
            pltpu.VMEM((RANK_K, B), F32),
        ],
    )(xsel, eqn, esk, respf, WihA, WihB, Whh, bsum, qW, qb, kW, kb, wv,
      emb_r, h0, c0)


def kernel(question, response, mask, q_neighbors, s_neighbors, qs_table,
           emb_q, emb_s, emb_r, ft_W, ft_b, agg_W, agg_b, last_W, last_b,
           Wih, Whh, bih, bhh, q_W, q_b, k_W, k_b, w_W, w_b):
    q = question.astype(jnp.int32)
    msk = mask.astype(jnp.int32)
    qn = q_neighbors.astype(jnp.int32)
    sn = s_neighbors.astype(jnp.int32)
    W0, W1, W2 = agg_W[0], agg_W[1], agg_W[2]
    b0 = agg_b[0].reshape(1, EMB)
    b1 = agg_b[1].reshape(1, EMB)
    b2 = agg_b[2].reshape(1, EMB)
    lb = last_b.reshape(1, EMB)
    fb = ft_b.reshape(1, EMB)
    es512 = jnp.pad(emb_s, ((0, NSP - NUM_S), (0, 0)))

    tall_a, t0a = _passA(qn, emb_q, es512, W2, b2, W0, b0)

    snp = jnp.pad(sn, ((0, NSP - NUM_S), (0, 0))).T.reshape(-1)
    idx1 = jnp.concatenate([snp, snp + NUM_Q])
    g1 = _sc_gather(tall_a.reshape(2 * NUM_Q, EMB), idx1, 4096, 128)

    t1a, t1b = _passB(es512, g1.reshape(8, NSP, EMB), W1, b1)

    tall = _passC(qn, emb_q, t0a, qs_table, t1a, t1b, emb_s,
                  W0, b0, last_W, lb, ft_W, fb)

    qT = q.T
    sel = (qT[:T] + NUM_Q * msk.T[:T]).reshape(-1)
    nxt = qT[1:].reshape(-1)
    idx2 = jnp.concatenate([sel, 2 * NUM_Q + nxt, 3 * NUM_Q + nxt])
    g2 = _sc_gather(tall.reshape(4 * NUM_Q, EMB), idx2, 3 * T * B, 96)
    g2 = g2.reshape(3, T, B, EMB)

    a = float(np.sqrt(6.0 / (B + EMB)))
    kh = jax.random.split(jax.random.key(42))
    h0 = jax.random.uniform(kh[0], (B, EMB), minval=-a, maxval=a, dtype=F32)
    c0 = jax.random.uniform(kh[1], (B, EMB), minval=-a, maxval=a, dtype=F32)

    respf = response.astype(F32).T[:T].reshape(T, B, 1)
    bsum = (bih + bhh).reshape(1, 4 * EMB)
    outp = _passR(g2[0], g2[1], g2[2], respf, Wih[:EMB], Wih[EMB:], Whh,
                  bsum, q_W, q_b.reshape(1, EMB), k_W, k_b.reshape(1, EMB),
                  w_W.reshape(2, EMB), emb_r, h0, c0)

    res = outp.reshape(T, B).T
    return jnp.concatenate([jnp.zeros((B, 1), F32), res], axis=1)


# speedup vs baseline: 12.0943x; 1.3629x over previous
import functools

import jax
import jax.numpy as jnp
import numpy as np
from jax import lax
from jax.experimental import pallas as pl
from jax.experimental.pallas import tpu as pltpu
from jax.experimental.pallas import tpu_sc as plsc

NUM_Q = 20000
NUM_S = 500
EMB = 128
B = 1024
S = 20
RANK_K = 10
T = S - 1
BQ = 512
NSP = 512
GRID_Q = (NUM_Q + BQ - 1) // BQ
NC, NS_SC = 2, 16
NW = NC * NS_SC
HI = lax.Precision.HIGHEST
F32 = jnp.float32


def _dot(a, b):
    return jnp.dot(a, b, preferred_element_type=F32, precision=HI)


def _dot_bt(a, b):
    return lax.dot_general(a, b, (((1,), (1,)), ((), ())),
                           preferred_element_type=F32, precision=HI)


def _sc_gather(table, idx, rows, chunk):
    per_w = rows // NW
    n_chunks = per_w // chunk
    mesh = plsc.VectorSubcoreMesh(core_axis_name="c", subcore_axis_name="s")

    @functools.partial(
        pl.kernel,
        out_type=jax.ShapeDtypeStruct((rows, EMB), F32),
        mesh=mesh,
        scratch_types=[
            pltpu.VMEM((chunk,), jnp.int32),
            pltpu.VMEM((chunk, EMB), F32),
            pltpu.SemaphoreType.DMA,
        ],
    )
    def gather(table_hbm, idx_hbm, out_hbm, idx_v, rows_v, sem):
        wid = lax.axis_index("s") * NC + lax.axis_index("c")
        base = wid * per_w
        for k in range(n_chunks):
            off = base + k * chunk
            pltpu.sync_copy(idx_hbm.at[pl.ds(off, chunk)], idx_v)
            pltpu.async_copy(table_hbm.at[idx_v], rows_v, sem).wait()
            pltpu.sync_copy(rows_v, out_hbm.at[pl.ds(off, chunk)])

    return gather(table, idx)


def _passA_body(qn_ref, eq_ref, es_ref, W2_ref, b2_ref, W0_ref, b0_ref,
                tall_ref, t0a_ref):
    qn = qn_ref[...]
    iot = lax.broadcasted_iota(jnp.int32, (BQ, NSP), 1)
    counts = jnp.zeros((BQ, NSP), F32)
    for j in range(4):
        counts += (qn[:, j:j + 1] == iot).astype(F32)
    qmean = _dot(counts, es_ref[...]) * 0.25
    x = eq_ref[...] + qmean
    tall_ref[0] = eq_ref[...]
    tall_ref[1] = jnp.tanh(_dot(x, W2_ref[...]) + b2_ref[...])
    t0a_ref[...] = jnp.tanh(_dot(x, W0_ref[...]) + b0_ref[...])


def _passA(qn, emb_q, es512, W2, b2, W0, b0):
    return pl.pallas_call(
        _passA_body,
        grid=(GRID_Q,),
        in_specs=[
            pl.BlockSpec((BQ, 4), lambda i: (i, 0)),
            pl.BlockSpec((BQ, EMB), lambda i: (i, 0)),
            pl.BlockSpec((NSP, EMB), lambda i: (0, 0)),
            pl.BlockSpec((EMB, EMB), lambda i: (0, 0)),
            pl.BlockSpec((1, EMB), lambda i: (0, 0)),
            pl.BlockSpec((EMB, EMB), lambda i: (0, 0)),
            pl.BlockSpec((1, EMB), lambda i: (0, 0)),
        ],
        out_specs=[
            pl.BlockSpec((2, BQ, EMB), lambda i: (0, i, 0)),
            pl.BlockSpec((BQ, EMB), lambda i: (i, 0)),
        ],
        out_shape=[
            jax.ShapeDtypeStruct((2, NUM_Q, EMB), F32),
            jax.ShapeDtypeStruct((NUM_Q, EMB), F32),
        ],
    )(qn, emb_q, es512, W2, b2, W0, b0)


def _passB_body(es_ref, g_ref, W1_ref, b1_ref, t1a_ref, t1b_ref):
    g = g_ref[...]
    sm0 = (g[0] + g[1] + g[2] + g[3]) * 0.25
    t1a = jnp.tanh(_dot(es_ref[...] + sm0, W1_ref[...]) + b1_ref[...])
    sm1 = (g[4] + g[5] + g[6] + g[7]) * 0.25
    t1b = jnp.tanh(_dot(t1a + sm1, W1_ref[...]) + b1_ref[...])
    t1a_ref[...] = t1a
    t1b_ref[...] = t1b


def _passB(es512, g, W1, b1):
    return pl.pallas_call(
        _passB_body,
        out_shape=[
            jax.ShapeDtypeStruct((NSP, EMB), F32),
            jax.ShapeDtypeStruct((NSP, EMB), F32),
        ],
    )(es512, g, W1, b1)


def _passC_body(qn_ref, eq_ref, t0a_ref, qs_ref, t1a_ref, t1b_ref, es_ref,
                W0_ref, b0_ref, lw_ref, lb_ref, fw_ref, fb_ref, tall_ref):
    qn = qn_ref[...]
    iot = lax.broadcasted_iota(jnp.int32, (BQ, NSP), 1)
    counts = jnp.zeros((BQ, NSP), F32)
    for j in range(4):
        counts += (qn[:, j:j + 1] == iot).astype(F32)
    qm1 = _dot(counts, t1a_ref[...]) * 0.25
    t0b = jnp.tanh(_dot(t0a_ref[...] + qm1, W0_ref[...]) + b0_ref[...])
    qm2 = _dot(counts, t1b_ref[...]) * 0.25
    t0c = jnp.tanh(_dot(t0b + qm2, W0_ref[...]) + b0_ref[...])
    qfin = jnp.tanh(_dot(t0c, lw_ref[...]) + lb_ref[...])
    tall_ref[0] = jnp.maximum(_dot(eq_ref[...], fw_ref[...]) + fb_ref[...], 0.0)
    tall_ref[1] = jnp.maximum(_dot(qfin, fw_ref[...]) + fb_ref[...], 0.0)
    tall_ref[2] = eq_ref[...]
    qs = qs_ref[...]
    esum = _dot(qs, es_ref[...])
    rs = jnp.sum(qs, axis=1, keepdims=True)
    tall_ref[3] = esum / jnp.maximum(rs, 1.0)


def _passC(qn, emb_q, t0a, qs_table, t1a, t1b, emb_s, W0, b0, lw, lb, fw, fb):
    return pl.pallas_call(
        _passC_body,
        grid=(GRID_Q,),
        in_specs=[
            pl.BlockSpec((BQ, 4), lambda i: (i, 0)),
            pl.BlockSpec((BQ, EMB), lambda i: (i, 0)),
            pl.BlockSpec((BQ, EMB), lambda i: (i, 0)),
            pl.BlockSpec((BQ, NUM_S), lambda i: (i, 0)),
            pl.BlockSpec((NSP, EMB), lambda i: (0, 0)),
            pl.BlockSpec((NSP, EMB), lambda i: (0, 0)),
            pl.BlockSpec((NUM_S, EMB), lambda i: (0, 0)),
            pl.BlockSpec((EMB, EMB), lambda i: (0, 0)),
            pl.BlockSpec((1, EMB), lambda i: (0, 0)),
            pl.BlockSpec((EMB, EMB), lambda i: (0, 0)),
            pl.BlockSpec((1, EMB), lambda i: (0, 0)),
            pl.BlockSpec((EMB, EMB), lambda i: (0, 0)),
            pl.BlockSpec((1, EMB), lambda i: (0, 0)),
        ],
        out_specs=pl.BlockSpec((4, BQ, EMB), lambda i: (0, i, 0)),
        out_shape=jax.ShapeDtypeStruct((4, NUM_Q, EMB), F32),
    )(qn, emb_q, t0a, qs_table, t1a, t1b, emb_s, W0, b0, lw, lb, fw, fb)


def _passR_body(xsel_ref, eqn_ref, esk_ref, resp_ref, WihAT_ref, WihBT_ref,
                WhhT_ref, bsumT_ref, qWT_ref, qbT_ref, kWT_ref, kbT_ref,
                w1_ref, w2_ref, er_ref, h0T_ref, c0T_ref, out_ref,
                hT_s, cT_s, qring, lqring):
    t = pl.program_id(0)

    @pl.when(t == 0)
    def _init():
        hT_s[...] = h0T_ref[...]
        cT_s[...] = c0T_ref[...]

    rWT = _dot_bt(WihBT_ref[...], er_ref[...])
    resp = resp_ref[0]
    gatesT = (_dot_bt(WihAT_ref[...], xsel_ref[0])
              + _dot(WhhT_ref[...], hT_s[...]) + bsumT_ref[...]
              + rWT[:, 0:1] + resp * (rWT[:, 1:2] - rWT[:, 0:1]))
    ig = jax.nn.sigmoid(gatesT[0:EMB])
    fg = jax.nn.sigmoid(gatesT[EMB:2 * EMB])
    gg = jnp.tanh(gatesT[2 * EMB:3 * EMB])
    og = jax.nn.sigmoid(gatesT[3 * EMB:4 * EMB])
    cT = fg * cT_s[...] + ig * gg
    hT = og * jnp.tanh(cT)
    cT_s[...] = cT
    hT_s[...] = hT

    qhT = _dot(qWT_ref[...], hT) + qbT_ref[...]
    lq_t = jnp.sum(qhT * w1_ref[...], axis=0, keepdims=True)
    slot = lax.rem(t, RANK_K)
    for s_i in range(RANK_K):
        @pl.when(slot == s_i)
        def _store(s_i=s_i):
            qring[s_i] = qhT
            lqring[s_i] = lq_t

    KmT0 = _dot_bt(kWT_ref[...], eqn_ref[0]) + kbT_ref[...]
    KmT1 = _dot_bt(kWT_ref[...], esk_ref[0]) + kbT_ref[...]
    w2 = w2_ref[...]
    lk0 = jnp.sum(KmT0 * w2, axis=0, keepdims=True)
    lk1 = jnp.sum(KmT1 * w2, axis=0, keepdims=True)

    ls, gs = [], []
    mx = jnp.full((1, B), -1e30, F32)
    for s_i in range(RANK_K):
        valid = jnp.logical_or(s_i <= t, t >= RANK_K)
        qrowT = qring[s_i]
        lq_s = lqring[s_i]
        for km, lk in ((KmT0, lk0), (KmT1, lk1)):
            l = jnp.where(valid, lq_s + lk, -1e30)
            g = jnp.where(
                valid,
                jax.nn.sigmoid(jnp.sum(qrowT * km, axis=0, keepdims=True)),
                0.0)
            ls.append(l)
            gs.append(g)
            mx = jnp.maximum(mx, l)
    num = jnp.zeros((1, B), F32)
    den = jnp.zeros((1, B), F32)
    for l, g in zip(ls, gs):
        e = jnp.exp(l - mx)
        num += e * g
        den += e
    out_ref[0] = num / den


def _passR(xsel, eqn, esk, respf, WihAT, WihBT, WhhT, bsumT, qWT, qbT,
           kWT, kbT, w1, w2, emb_r, h0T, c0T):
    full = lambda shape: pl.BlockSpec(shape, lambda t: tuple(0 for _ in shape))
    return pl.pallas_call(
        _passR_body,
        grid=(T,),
        in_specs=[
            pl.BlockSpec((1, B, EMB), lambda t: (t, 0, 0)),
            pl.BlockSpec((1, B, EMB), lambda t: (t, 0, 0)),
            pl.BlockSpec((1, B, EMB), lambda t: (t, 0, 0)),
            pl.BlockSpec((1, 1, B), lambda t: (t, 0, 0)),
            full((4 * EMB, EMB)),
            full((4 * EMB, EMB)),
            full((4 * EMB, EMB)),
            full((4 * EMB, 1)),
            full((EMB, EMB)),
            full((EMB, 1)),
            full((EMB, EMB)),
            full((EMB, 1)),
            full((EMB, 1)),
            full((EMB, 1)),
            full((2, EMB)),
            full((EMB, B)),
            full((EMB, B)),
        ],
        out_specs=pl.BlockSpec((1, 1, B), lambda t: (t, 0, 0)),
        out_shape=jax.ShapeDtypeStruct((T, 1, B), F32),
        scratch_shapes=[
            pltpu.VMEM((EMB, B), F32),
            pltpu.VMEM((EMB, B), F32),
            pltpu.VMEM((RANK_K, EMB, B), F32),
            pltpu.VMEM((RANK_K, 1, B), F32),
        ],
    )(xsel, eqn, esk, respf, WihAT, WihBT, WhhT, bsumT, qWT, qbT, kWT, kbT,
      w1, w2, emb_r, h0T, c0T)


def kernel(question, response, mask, q_neighbors, s_neighbors, qs_table,
           emb_q, emb_s, emb_r, ft_W, ft_b, agg_W, agg_b, last_W, last_b,
           Wih, Whh, bih, bhh, q_W, q_b, k_W, k_b, w_W, w_b):
    q = question.astype(jnp.int32)
    msk = mask.astype(jnp.int32)
    qn = q_neighbors.astype(jnp.int32)
    sn = s_neighbors.astype(jnp.int32)
    W0, W1, W2 = agg_W[0], agg_W[1], agg_W[2]
    b0 = agg_b[0].reshape(1, EMB)
    b1 = agg_b[1].reshape(1, EMB)
    b2 = agg_b[2].reshape(1, EMB)
    lb = last_b.reshape(1, EMB)
    fb = ft_b.reshape(1, EMB)
    es512 = jnp.pad(emb_s, ((0, NSP - NUM_S), (0, 0)))

    tall_a, t0a = _passA(qn, emb_q, es512, W2, b2, W0, b0)

    snp = jnp.pad(sn, ((0, NSP - NUM_S), (0, 0))).T.reshape(-1)
    idx1 = jnp.concatenate([snp, snp + NUM_Q])
    g1 = _sc_gather(tall_a.reshape(2 * NUM_Q, EMB), idx1, 4096, 128)

    t1a, t1b = _passB(es512, g1.reshape(8, NSP, EMB), W1, b1)

    tall = _passC(qn, emb_q, t0a, qs_table, t1a, t1b, emb_s,
                  W0, b0, last_W, lb, ft_W, fb)

    qT = q.T
    sel = (qT[:T] + NUM_Q * msk.T[:T]).reshape(-1)
    nxt = qT[1:].reshape(-1)
    idx2 = jnp.concatenate([sel, 2 * NUM_Q + nxt, 3 * NUM_Q + nxt])
    g2 = _sc_gather(tall.reshape(4 * NUM_Q, EMB), idx2, 3 * T * B, 96)
    g2 = g2.reshape(3, T, B, EMB)

    a = float(np.sqrt(6.0 / (B + EMB)))
    kh = jax.random.split(jax.random.key(42))
    h0 = jax.random.uniform(kh[0], (B, EMB), minval=-a, maxval=a, dtype=F32)
    c0 = jax.random.uniform(kh[1], (B, EMB), minval=-a, maxval=a, dtype=F32)

    respf = response.astype(F32).T[:T].reshape(T, 1, B)
    bsumT = (bih + bhh).reshape(4 * EMB, 1)
    outp = _passR(g2[0], g2[1], g2[2], respf, Wih[:EMB].T, Wih[EMB:].T,
                  Whh.T, bsumT, q_W.T, q_b.reshape(EMB, 1), k_W.T,
                  k_b.reshape(EMB, 1), w_W[:EMB], w_W[EMB:], emb_r,
                  h0.T, c0.T)

    res = outp.reshape(T, B).T
    return jnp.concatenate([jnp.zeros((B, 1), F32), res], axis=1)


# speedup vs baseline: 23.7572x; 1.9643x over previous
import functools

import jax
import jax.numpy as jnp
import numpy as np
from jax import lax
from jax.experimental import pallas as pl
from jax.experimental.pallas import tpu as pltpu
from jax.experimental.pallas import tpu_sc as plsc

NUM_Q = 20000
NUM_S = 500
EMB = 128
B = 1024
S = 20
RANK_K = 10
T = S - 1
BQ = 512
NSP = 512
GRID_Q = (NUM_Q + BQ - 1) // BQ
NC, NS_SC = 2, 16
NW = NC * NS_SC
HI = lax.Precision.HIGHEST
F32 = jnp.float32


def _dot(a, b):
    return jnp.dot(a, b, preferred_element_type=F32)


def _dot_bt(a, b):
    return lax.dot_general(a, b, (((1,), (1,)), ((), ())),
                           preferred_element_type=F32)


def _sc_gather(table, idx, rows, chunk):
    per_w = rows // NW
    n_chunks = per_w // chunk
    mesh = plsc.VectorSubcoreMesh(core_axis_name="c", subcore_axis_name="s")

    @functools.partial(
        pl.kernel,
        out_type=jax.ShapeDtypeStruct((rows, EMB), F32),
        mesh=mesh,
        scratch_types=[
            pltpu.VMEM((chunk,), jnp.int32),
            pltpu.VMEM((chunk, EMB), F32),
            pltpu.SemaphoreType.DMA,
        ],
    )
    def gather(table_hbm, idx_hbm, out_hbm, idx_v, rows_v, sem):
        wid = lax.axis_index("s") * NC + lax.axis_index("c")
        base = wid * per_w
        for k in range(n_chunks):
            off = base + k * chunk
            pltpu.sync_copy(idx_hbm.at[pl.ds(off, chunk)], idx_v)
            pltpu.async_copy(table_hbm.at[idx_v], rows_v, sem).wait()
            pltpu.sync_copy(rows_v, out_hbm.at[pl.ds(off, chunk)])

    return gather(table, idx)


def _passA_body(qn_ref, eq_ref, es_ref, W2_ref, b2_ref, W0_ref, b0_ref,
                tall_ref, t0a_ref):
    qn = qn_ref[...]
    iot = lax.broadcasted_iota(jnp.int32, (BQ, NSP), 1)
    counts = jnp.zeros((BQ, NSP), jnp.bfloat16)
    for j in range(4):
        counts += (qn[:, j:j + 1] == iot).astype(jnp.bfloat16)
    qmean = _dot(counts, es_ref[...].astype(jnp.bfloat16)) * 0.25
    x = eq_ref[...] + qmean
    tall_ref[0] = eq_ref[...]
    tall_ref[1] = jnp.tanh(_dot(x, W2_ref[...]) + b2_ref[...])
    t0a_ref[...] = jnp.tanh(_dot(x, W0_ref[...]) + b0_ref[...])


def _passA(qn, emb_q, es512, W2, b2, W0, b0):
    return pl.pallas_call(
        _passA_body,
        grid=(GRID_Q,),
        in_specs=[
            pl.BlockSpec((BQ, 4), lambda i: (i, 0)),
            pl.BlockSpec((BQ, EMB), lambda i: (i, 0)),
            pl.BlockSpec((NSP, EMB), lambda i: (0, 0)),
            pl.BlockSpec((EMB, EMB), lambda i: (0, 0)),
            pl.BlockSpec((1, EMB), lambda i: (0, 0)),
            pl.BlockSpec((EMB, EMB), lambda i: (0, 0)),
            pl.BlockSpec((1, EMB), lambda i: (0, 0)),
        ],
        out_specs=[
            pl.BlockSpec((2, BQ, EMB), lambda i: (0, i, 0)),
            pl.BlockSpec((BQ, EMB), lambda i: (i, 0)),
        ],
        out_shape=[
            jax.ShapeDtypeStruct((2, NUM_Q, EMB), F32),
            jax.ShapeDtypeStruct((NUM_Q, EMB), F32),
        ],
    )(qn, emb_q, es512, W2, b2, W0, b0)


def _passB_body(es_ref, g_ref, W1_ref, b1_ref, t1a_ref, t1b_ref):
    g = g_ref[...]
    sm0 = (g[0] + g[1] + g[2] + g[3]) * 0.25
    t1a = jnp.tanh(_dot(es_ref[...] + sm0, W1_ref[...]) + b1_ref[...])
    sm1 = (g[4] + g[5] + g[6] + g[7]) * 0.25
    t1b = jnp.tanh(_dot(t1a + sm1, W1_ref[...]) + b1_ref[...])
    t1a_ref[...] = t1a
    t1b_ref[...] = t1b


def _passB(es512, g, W1, b1):
    return pl.pallas_call(
        _passB_body,
        out_shape=[
            jax.ShapeDtypeStruct((NSP, EMB), F32),
            jax.ShapeDtypeStruct((NSP, EMB), F32),
        ],
    )(es512, g, W1, b1)


def _passC_body(qn_ref, eq_ref, t0a_ref, qs_ref, t1a_ref, t1b_ref, es_ref,
                W0_ref, b0_ref, lw_ref, lb_ref, fw_ref, fb_ref, tall_ref):
    qn = qn_ref[...]
    iot = lax.broadcasted_iota(jnp.int32, (BQ, NSP), 1)
    counts = jnp.zeros((BQ, NSP), jnp.bfloat16)
    for j in range(4):
        counts += (qn[:, j:j + 1] == iot).astype(jnp.bfloat16)
    qm1 = _dot(counts, t1a_ref[...].astype(jnp.bfloat16)) * 0.25
    t0b = jnp.tanh(_dot(t0a_ref[...] + qm1, W0_ref[...]) + b0_ref[...])
    qm2 = _dot(counts, t1b_ref[...].astype(jnp.bfloat16)) * 0.25
    t0c = jnp.tanh(_dot(t0b + qm2, W0_ref[...]) + b0_ref[...])
    qfin = jnp.tanh(_dot(t0c, lw_ref[...]) + lb_ref[...])
    tall_ref[0] = jnp.maximum(_dot(eq_ref[...], fw_ref[...]) + fb_ref[...], 0.0)
    tall_ref[1] = jnp.maximum(_dot(qfin, fw_ref[...]) + fb_ref[...], 0.0)
    tall_ref[2] = eq_ref[...]
    qs = qs_ref[...]
    esum = _dot(qs.astype(jnp.bfloat16), es_ref[...].astype(jnp.bfloat16))
    rs = jnp.sum(qs, axis=1, keepdims=True)
    tall_ref[3] = esum / jnp.maximum(rs, 1.0)


def _passC(qn, emb_q, t0a, qs_table, t1a, t1b, emb_s, W0, b0, lw, lb, fw, fb):
    return pl.pallas_call(
        _passC_body,
        grid=(GRID_Q,),
        in_specs=[
            pl.BlockSpec((BQ, 4), lambda i: (i, 0)),
            pl.BlockSpec((BQ, EMB), lambda i: (i, 0)),
            pl.BlockSpec((BQ, EMB), lambda i: (i, 0)),
            pl.BlockSpec((BQ, NUM_S), lambda i: (i, 0)),
            pl.BlockSpec((NSP, EMB), lambda i: (0, 0)),
            pl.BlockSpec((NSP, EMB), lambda i: (0, 0)),
            pl.BlockSpec((NUM_S, EMB), lambda i: (0, 0)),
            pl.BlockSpec((EMB, EMB), lambda i: (0, 0)),
            pl.BlockSpec((1, EMB), lambda i: (0, 0)),
            pl.BlockSpec((EMB, EMB), lambda i: (0, 0)),
            pl.BlockSpec((1, EMB), lambda i: (0, 0)),
            pl.BlockSpec((EMB, EMB), lambda i: (0, 0)),
            pl.BlockSpec((1, EMB), lambda i: (0, 0)),
        ],
        out_specs=pl.BlockSpec((4, BQ, EMB), lambda i: (0, i, 0)),
        out_shape=jax.ShapeDtypeStruct((4, NUM_Q, EMB), F32),
    )(qn, emb_q, t0a, qs_table, t1a, t1b, emb_s, W0, b0, lw, lb, fw, fb)


def _passR_body(xsel_ref, eqn_ref, esk_ref, resp_ref, WihAT_ref, WihBT_ref,
                WhhT_ref, bsumT_ref, qWT_ref, qbT_ref, kWT_ref, kbT_ref,
                w1_ref, w2_ref, er_ref, h0T_ref, c0T_ref, out_ref,
                hT_s, cT_s, qring, lqring):
    t = pl.program_id(0)

    @pl.when(t == 0)
    def _init():
        hT_s[...] = h0T_ref[...]
        cT_s[...] = c0T_ref[...]

    rWT = _dot_bt(WihBT_ref[...], er_ref[...])
    resp = resp_ref[0]
    gatesT = (_dot_bt(WihAT_ref[...], xsel_ref[0])
              + _dot(WhhT_ref[...], hT_s[...]) + bsumT_ref[...]
              + rWT[:, 0:1] + resp * (rWT[:, 1:2] - rWT[:, 0:1]))
    ig = jax.nn.sigmoid(gatesT[0:EMB])
    fg = jax.nn.sigmoid(gatesT[EMB:2 * EMB])
    gg = jnp.tanh(gatesT[2 * EMB:3 * EMB])
    og = jax.nn.sigmoid(gatesT[3 * EMB:4 * EMB])
    cT = fg * cT_s[...] + ig * gg
    hT = og * jnp.tanh(cT)
    cT_s[...] = cT
    hT_s[...] = hT

    qhT = _dot(qWT_ref[...], hT) + qbT_ref[...]
    lq_t = jnp.sum(qhT * w1_ref[...], axis=0, keepdims=True)
    slot = lax.rem(t, RANK_K)
    for s_i in range(RANK_K):
        @pl.when(slot == s_i)
        def _store(s_i=s_i):
            qring[s_i] = qhT
            lqring[s_i] = lq_t

    KmT0 = _dot_bt(kWT_ref[...], eqn_ref[0]) + kbT_ref[...]
    KmT1 = _dot_bt(kWT_ref[...], esk_ref[0]) + kbT_ref[...]
    w2 = w2_ref[...]
    lk0 = jnp.sum(KmT0 * w2, axis=0, keepdims=True)
    lk1 = jnp.sum(KmT1 * w2, axis=0, keepdims=True)

    ls, gs = [], []
    mx = jnp.full((1, B), -1e30, F32)
    for s_i in range(RANK_K):
        valid = jnp.logical_or(s_i <= t, t >= RANK_K)
        qrowT = qring[s_i]
        lq_s = lqring[s_i]
        for km, lk in ((KmT0, lk0), (KmT1, lk1)):
            l = jnp.where(valid, lq_s + lk, -1e30)
            g = jnp.where(
                valid,
                jax.nn.sigmoid(jnp.sum(qrowT * km, axis=0, keepdims=True)),
                0.0)
            ls.append(l)
            gs.append(g)
            mx = jnp.maximum(mx, l)
    num = jnp.zeros((1, B), F32)
    den = jnp.zeros((1, B), F32)
    for l, g in zip(ls, gs):
        e = jnp.exp(l - mx)
        num += e * g
        den += e
    out_ref[0] = num / den


def _passR(xsel, eqn, esk, respf, WihAT, WihBT, WhhT, bsumT, qWT, qbT,
           kWT, kbT, w1, w2, emb_r, h0T, c0T):
    full = lambda shape: pl.BlockSpec(shape, lambda t: tuple(0 for _ in shape))
    return pl.pallas_call(
        _passR_body,
        grid=(T,),
        in_specs=[
            pl.BlockSpec((1, B, EMB), lambda t: (t, 0, 0)),
            pl.BlockSpec((1, B, EMB), lambda t: (t, 0, 0)),
            pl.BlockSpec((1, B, EMB), lambda t: (t, 0, 0)),
            pl.BlockSpec((1, 1, B), lambda t: (t, 0, 0)),
            full((4 * EMB, EMB)),
            full((4 * EMB, EMB)),
            full((4 * EMB, EMB)),
            full((4 * EMB, 1)),
            full((EMB, EMB)),
            full((EMB, 1)),
            full((EMB, EMB)),
            full((EMB, 1)),
            full((EMB, 1)),
            full((EMB, 1)),
            full((2, EMB)),
            full((EMB, B)),
            full((EMB, B)),
        ],
        out_specs=pl.BlockSpec((1, 1, B), lambda t: (t, 0, 0)),
        out_shape=jax.ShapeDtypeStruct((T, 1, B), F32),
        scratch_shapes=[
            pltpu.VMEM((EMB, B), F32),
            pltpu.VMEM((EMB, B), F32),
            pltpu.VMEM((RANK_K, EMB, B), F32),
            pltpu.VMEM((RANK_K, 1, B), F32),
        ],
    )(xsel, eqn, esk, respf, WihAT, WihBT, WhhT, bsumT, qWT, qbT, kWT, kbT,
      w1, w2, emb_r, h0T, c0T)


def kernel(question, response, mask, q_neighbors, s_neighbors, qs_table,
           emb_q, emb_s, emb_r, ft_W, ft_b, agg_W, agg_b, last_W, last_b,
           Wih, Whh, bih, bhh, q_W, q_b, k_W, k_b, w_W, w_b):
    q = question.astype(jnp.int32)
    msk = mask.astype(jnp.int32)
    qn = q_neighbors.astype(jnp.int32)
    sn = s_neighbors.astype(jnp.int32)
    W0, W1, W2 = agg_W[0], agg_W[1], agg_W[2]
    b0 = agg_b[0].reshape(1, EMB)
    b1 = agg_b[1].reshape(1, EMB)
    b2 = agg_b[2].reshape(1, EMB)
    lb = last_b.reshape(1, EMB)
    fb = ft_b.reshape(1, EMB)
    es512 = jnp.pad(emb_s, ((0, NSP - NUM_S), (0, 0)))

    tall_a, t0a = _passA(qn, emb_q, es512, W2, b2, W0, b0)

    snp = jnp.pad(sn, ((0, NSP - NUM_S), (0, 0))).T.reshape(-1)
    idx1 = jnp.concatenate([snp, snp + NUM_Q])
    g1 = _sc_gather(tall_a.reshape(2 * NUM_Q, EMB), idx1, 4096, 128)

    t1a, t1b = _passB(es512, g1.reshape(8, NSP, EMB), W1, b1)

    tall = _passC(qn, emb_q, t0a, qs_table, t1a, t1b, emb_s,
                  W0, b0, last_W, lb, ft_W, fb)

    qT = q.T
    sel = (qT[:T] + NUM_Q * msk.T[:T]).reshape(-1)
    nxt = qT[1:].reshape(-1)
    idx2 = jnp.concatenate([sel, 2 * NUM_Q + nxt, 3 * NUM_Q + nxt])
    g2 = _sc_gather(tall.reshape(4 * NUM_Q, EMB), idx2, 3 * T * B, 96)
    g2 = g2.reshape(3, T, B, EMB)

    a = float(np.sqrt(6.0 / (B + EMB)))
    kh = jax.random.split(jax.random.key(42))
    h0 = jax.random.uniform(kh[0], (B, EMB), minval=-a, maxval=a, dtype=F32)
    c0 = jax.random.uniform(kh[1], (B, EMB), minval=-a, maxval=a, dtype=F32)

    respf = response.astype(F32).T[:T].reshape(T, 1, B)
    bsumT = (bih + bhh).reshape(4 * EMB, 1)
    outp = _passR(g2[0], g2[1], g2[2], respf, Wih[:EMB].T, Wih[EMB:].T,
                  Whh.T, bsumT, q_W.T, q_b.reshape(EMB, 1), k_W.T,
                  k_b.reshape(EMB, 1), w_W[:EMB], w_W[EMB:], emb_r,
                  h0.T, c0.T)

    res = outp.reshape(T, B).T
    return jnp.concatenate([jnp.zeros((B, 1), F32), res], axis=1)


# speedup vs baseline: 25.2106x; 1.0612x over previous
import functools

import jax
import jax.numpy as jnp
import numpy as np
from jax import lax
from jax.experimental import pallas as pl
from jax.experimental.pallas import tpu as pltpu
from jax.experimental.pallas import tpu_sc as plsc

NUM_Q = 20000
NUM_S = 500
EMB = 128
B = 1024
S = 20
RANK_K = 10
T = S - 1
BQ = 512
NSP = 512
GRID_Q = (NUM_Q + BQ - 1) // BQ
NC, NS_SC = 2, 16
NW = NC * NS_SC
HI = lax.Precision.HIGHEST
F32 = jnp.float32


def _dot(a, b):
    return jnp.dot(a, b, preferred_element_type=F32)


def _dot_bt(a, b):
    return lax.dot_general(a, b, (((1,), (1,)), ((), ())),
                           preferred_element_type=F32)


def _sc_gather(table, idx, rows, chunk):
    per_w = rows // NW
    n_chunks = per_w // chunk
    nb = min(3, n_chunks)
    mesh = plsc.VectorSubcoreMesh(core_axis_name="c", subcore_axis_name="s")

    @functools.partial(
        pl.kernel,
        out_type=jax.ShapeDtypeStruct((rows, EMB), F32),
        mesh=mesh,
        scratch_types=(
            [pltpu.VMEM((chunk,), jnp.int32) for _ in range(nb)]
            + [pltpu.VMEM((chunk, EMB), F32) for _ in range(nb)]
            + [pltpu.SemaphoreType.DMA, pltpu.SemaphoreType.DMA]
        ),
    )
    def gather(table_hbm, idx_hbm, out_hbm, *rest):
        ibufs = rest[:nb]
        bufs = rest[nb:2 * nb]
        gsem, wsem = rest[2 * nb], rest[2 * nb + 1]
        wid = lax.axis_index("s") * NC + lax.axis_index("c")
        base = wid * per_w

        def start(k):
            pltpu.sync_copy(idx_hbm.at[pl.ds(base + k * chunk, chunk)],
                            ibufs[k % nb])
            return pltpu.async_copy(table_hbm.at[ibufs[k % nb]],
                                    bufs[k % nb], gsem)

        gps = {0: start(0)}
        wps = {}
        for k in range(n_chunks):
            if k + 1 < n_chunks:
                if k + 1 >= nb:
                    wps[k + 1 - nb].wait()
                gps[k + 1] = start(k + 1)
            gps[k].wait()
            wps[k] = pltpu.async_copy(
                bufs[k % nb], out_hbm.at[pl.ds(base + k * chunk, chunk)], wsem)
        for k in range(max(0, n_chunks - nb), n_chunks):
            wps[k].wait()

    return gather(table, idx)


def _passA_body(qn_ref, eq_ref, es_ref, W2_ref, b2_ref, W0_ref, b0_ref,
                tall_ref, t0a_ref):
    qn = qn_ref[...]
    iot = lax.broadcasted_iota(jnp.int32, (BQ, NSP), 1)
    counts = jnp.zeros((BQ, NSP), jnp.bfloat16)
    for j in range(4):
        counts += (qn[:, j:j + 1] == iot).astype(jnp.bfloat16)
    qmean = _dot(counts, es_ref[...].astype(jnp.bfloat16)) * 0.25
    x = eq_ref[...] + qmean
    tall_ref[0] = eq_ref[...]
    tall_ref[1] = jnp.tanh(_dot(x, W2_ref[...]) + b2_ref[...])
    t0a_ref[...] = jnp.tanh(_dot(x, W0_ref[...]) + b0_ref[...])


def _passA(qn, emb_q, es512, W2, b2, W0, b0):
    return pl.pallas_call(
        _passA_body,
        grid=(GRID_Q,),
        in_specs=[
            pl.BlockSpec((BQ, 4), lambda i: (i, 0)),
            pl.BlockSpec((BQ, EMB), lambda i: (i, 0)),
            pl.BlockSpec((NSP, EMB), lambda i: (0, 0)),
            pl.BlockSpec((EMB, EMB), lambda i: (0, 0)),
            pl.BlockSpec((1, EMB), lambda i: (0, 0)),
            pl.BlockSpec((EMB, EMB), lambda i: (0, 0)),
            pl.BlockSpec((1, EMB), lambda i: (0, 0)),
        ],
        out_specs=[
            pl.BlockSpec((2, BQ, EMB), lambda i: (0, i, 0)),
            pl.BlockSpec((BQ, EMB), lambda i: (i, 0)),
        ],
        out_shape=[
            jax.ShapeDtypeStruct((2, NUM_Q, EMB), F32),
            jax.ShapeDtypeStruct((NUM_Q, EMB), F32),
        ],
    )(qn, emb_q, es512, W2, b2, W0, b0)


def _passB_body(es_ref, g_ref, W1_ref, b1_ref, t1a_ref, t1b_ref):
    g = g_ref[...]
    sm0 = (g[0] + g[1] + g[2] + g[3]) * 0.25
    t1a = jnp.tanh(_dot(es_ref[...] + sm0, W1_ref[...]) + b1_ref[...])
    sm1 = (g[4] + g[5] + g[6] + g[7]) * 0.25
    t1b = jnp.tanh(_dot(t1a + sm1, W1_ref[...]) + b1_ref[...])
    t1a_ref[...] = t1a
    t1b_ref[...] = t1b


def _passB(es512, g, W1, b1):
    return pl.pallas_call(
        _passB_body,
        out_shape=[
            jax.ShapeDtypeStruct((NSP, EMB), F32),
            jax.ShapeDtypeStruct((NSP, EMB), F32),
        ],
    )(es512, g, W1, b1)


def _passC_body(qn_ref, eq_ref, t0a_ref, qs_ref, t1a_ref, t1b_ref, es_ref,
                W0_ref, b0_ref, lw_ref, lb_ref, fw_ref, fb_ref, tall_ref):
    qn = qn_ref[...]
    iot = lax.broadcasted_iota(jnp.int32, (BQ, NSP), 1)
    counts = jnp.zeros((BQ, NSP), jnp.bfloat16)
    for j in range(4):
        counts += (qn[:, j:j + 1] == iot).astype(jnp.bfloat16)
    qm1 = _dot(counts, t1a_ref[...].astype(jnp.bfloat16)) * 0.25
    t0b = jnp.tanh(_dot(t0a_ref[...] + qm1, W0_ref[...]) + b0_ref[...])
    qm2 = _dot(counts, t1b_ref[...].astype(jnp.bfloat16)) * 0.25
    t0c = jnp.tanh(_dot(t0b + qm2, W0_ref[...]) + b0_ref[...])
    qfin = jnp.tanh(_dot(t0c, lw_ref[...]) + lb_ref[...])
    tall_ref[0] = jnp.maximum(_dot(eq_ref[...], fw_ref[...]) + fb_ref[...], 0.0)
    tall_ref[1] = jnp.maximum(_dot(qfin, fw_ref[...]) + fb_ref[...], 0.0)
    tall_ref[2] = eq_ref[...]
    qs = qs_ref[...]
    esum = _dot(qs.astype(jnp.bfloat16), es_ref[...].astype(jnp.bfloat16))
    rs = jnp.sum(qs, axis=1, keepdims=True)
    tall_ref[3] = esum / jnp.maximum(rs, 1.0)


def _passC(qn, emb_q, t0a, qs_table, t1a, t1b, emb_s, W0, b0, lw, lb, fw, fb):
    return pl.pallas_call(
        _passC_body,
        grid=(GRID_Q,),
        in_specs=[
            pl.BlockSpec((BQ, 4), lambda i: (i, 0)),
            pl.BlockSpec((BQ, EMB), lambda i: (i, 0)),
            pl.BlockSpec((BQ, EMB), lambda i: (i, 0)),
            pl.BlockSpec((BQ, NUM_S), lambda i: (i, 0)),
            pl.BlockSpec((NSP, EMB), lambda i: (0, 0)),
            pl.BlockSpec((NSP, EMB), lambda i: (0, 0)),
            pl.BlockSpec((NUM_S, EMB), lambda i: (0, 0)),
            pl.BlockSpec((EMB, EMB), lambda i: (0, 0)),
            pl.BlockSpec((1, EMB), lambda i: (0, 0)),
            pl.BlockSpec((EMB, EMB), lambda i: (0, 0)),
            pl.BlockSpec((1, EMB), lambda i: (0, 0)),
            pl.BlockSpec((EMB, EMB), lambda i: (0, 0)),
            pl.BlockSpec((1, EMB), lambda i: (0, 0)),
        ],
        out_specs=pl.BlockSpec((4, BQ, EMB), lambda i: (0, i, 0)),
        out_shape=jax.ShapeDtypeStruct((4, NUM_Q, EMB), F32),
    )(qn, emb_q, t0a, qs_table, t1a, t1b, emb_s, W0, b0, lw, lb, fw, fb)


def _passR_body(xsel_ref, eqn_ref, esk_ref, resp_ref, WihAT_ref, WihBT_ref,
                WhhT_ref, bsumT_ref, qWT_ref, qbT_ref, kWT_ref, kbT_ref,
                w1_ref, w2_ref, er_ref, h0T_ref, c0T_ref, out_ref,
                hT_s, cT_s, qring, lqring):
    t = pl.program_id(0)

    @pl.when(t == 0)
    def _init():
        hT_s[...] = h0T_ref[...]
        cT_s[...] = c0T_ref[...]

    rWT = _dot_bt(WihBT_ref[...], er_ref[...])
    resp = resp_ref[0]
    gatesT = (_dot_bt(WihAT_ref[...], xsel_ref[0])
              + _dot(WhhT_ref[...], hT_s[...]) + bsumT_ref[...]
              + rWT[:, 0:1] + resp * (rWT[:, 1:2] - rWT[:, 0:1]))
    ig = jax.nn.sigmoid(gatesT[0:EMB])
    fg = jax.nn.sigmoid(gatesT[EMB:2 * EMB])
    gg = jnp.tanh(gatesT[2 * EMB:3 * EMB])
    og = jax.nn.sigmoid(gatesT[3 * EMB:4 * EMB])
    cT = fg * cT_s[...] + ig * gg
    hT = og * jnp.tanh(cT)
    cT_s[...] = cT
    hT_s[...] = hT

    qhT = _dot(qWT_ref[...], hT) + qbT_ref[...]
    lq_t = jnp.sum(qhT * w1_ref[...], axis=0, keepdims=True)
    slot = lax.rem(t, RANK_K)
    for s_i in range(RANK_K):
        @pl.when(slot == s_i)
        def _store(s_i=s_i):
            qring[s_i] = qhT
            lqring[s_i] = lq_t

    KmT0 = _dot_bt(kWT_ref[...], eqn_ref[0]) + kbT_ref[...]
    KmT1 = _dot_bt(kWT_ref[...], esk_ref[0]) + kbT_ref[...]
    w2 = w2_ref[...]
    lk0 = jnp.sum(KmT0 * w2, axis=0, keepdims=True)
    lk1 = jnp.sum(KmT1 * w2, axis=0, keepdims=True)

    ls, gs = [], []
    mx = jnp.full((1, B), -1e30, F32)
    for s_i in range(RANK_K):
        valid = jnp.logical_or(s_i <= t, t >= RANK_K)
        qrowT = qring[s_i]
        lq_s = lqring[s_i]
        for km, lk in ((KmT0, lk0), (KmT1, lk1)):
            l = jnp.where(valid, lq_s + lk, -1e30)
            g = jnp.where(
                valid,
                jax.nn.sigmoid(jnp.sum(qrowT * km, axis=0, keepdims=True)),
                0.0)
            ls.append(l)
            gs.append(g)
            mx = jnp.maximum(mx, l)
    num = jnp.zeros((1, B), F32)
    den = jnp.zeros((1, B), F32)
    for l, g in zip(ls, gs):
        e = jnp.exp(l - mx)
        num += e * g
        den += e
    out_ref[0] = num / den


def _passR(xsel, eqn, esk, respf, WihAT, WihBT, WhhT, bsumT, qWT, qbT,
           kWT, kbT, w1, w2, emb_r, h0T, c0T):
    full = lambda shape: pl.BlockSpec(shape, lambda t: tuple(0 for _ in shape))
    return pl.pallas_call(
        _passR_body,
        grid=(T,),
        in_specs=[
            pl.BlockSpec((1, B, EMB), lambda t: (t, 0, 0)),
            pl.BlockSpec((1, B, EMB), lambda t: (t, 0, 0)),
            pl.BlockSpec((1, B, EMB), lambda t: (t, 0, 0)),
            pl.BlockSpec((1, 1, B), lambda t: (t, 0, 0)),
            full((4 * EMB, EMB)),
            full((4 * EMB, EMB)),
            full((4 * EMB, EMB)),
            full((4 * EMB, 1)),
            full((EMB, EMB)),
            full((EMB, 1)),
            full((EMB, EMB)),
            full((EMB, 1)),
            full((EMB, 1)),
            full((EMB, 1)),
            full((2, EMB)),
            full((EMB, B)),
            full((EMB, B)),
        ],
        out_specs=pl.BlockSpec((1, 1, B), lambda t: (t, 0, 0)),
        out_shape=jax.ShapeDtypeStruct((T, 1, B), F32),
        scratch_shapes=[
            pltpu.VMEM((EMB, B), F32),
            pltpu.VMEM((EMB, B), F32),
            pltpu.VMEM((RANK_K, EMB, B), F32),
            pltpu.VMEM((RANK_K, 1, B), F32),
        ],
    )(xsel, eqn, esk, respf, WihAT, WihBT, WhhT, bsumT, qWT, qbT, kWT, kbT,
      w1, w2, emb_r, h0T, c0T)


def kernel(question, response, mask, q_neighbors, s_neighbors, qs_table,
           emb_q, emb_s, emb_r, ft_W, ft_b, agg_W, agg_b, last_W, last_b,
           Wih, Whh, bih, bhh, q_W, q_b, k_W, k_b, w_W, w_b):
    q = question.astype(jnp.int32)
    msk = mask.astype(jnp.int32)
    qn = q_neighbors.astype(jnp.int32)
    sn = s_neighbors.astype(jnp.int32)
    W0, W1, W2 = agg_W[0], agg_W[1], agg_W[2]
    b0 = agg_b[0].reshape(1, EMB)
    b1 = agg_b[1].reshape(1, EMB)
    b2 = agg_b[2].reshape(1, EMB)
    lb = last_b.reshape(1, EMB)
    fb = ft_b.reshape(1, EMB)
    es512 = jnp.pad(emb_s, ((0, NSP - NUM_S), (0, 0)))

    tall_a, t0a = _passA(qn, emb_q, es512, W2, b2, W0, b0)

    snp = jnp.pad(sn, ((0, NSP - NUM_S), (0, 0))).T.reshape(-1)
    idx1 = jnp.concatenate([snp, snp + NUM_Q])
    g1 = _sc_gather(tall_a.reshape(2 * NUM_Q, EMB), idx1, 4096, 128)

    t1a, t1b = _passB(es512, g1.reshape(8, NSP, EMB), W1, b1)

    tall = _passC(qn, emb_q, t0a, qs_table, t1a, t1b, emb_s,
                  W0, b0, last_W, lb, ft_W, fb)

    qT = q.T
    sel = (qT[:T] + NUM_Q * msk.T[:T]).reshape(-1)
    nxt = qT[1:].reshape(-1)
    idx2 = jnp.concatenate([sel, 2 * NUM_Q + nxt, 3 * NUM_Q + nxt])
    g2 = _sc_gather(tall.reshape(4 * NUM_Q, EMB), idx2, 3 * T * B, 96)
    g2 = g2.reshape(3, T, B, EMB)

    a = float(np.sqrt(6.0 / (B + EMB)))
    kh = jax.random.split(jax.random.key(42))
    h0 = jax.random.uniform(kh[0], (B, EMB), minval=-a, maxval=a, dtype=F32)
    c0 = jax.random.uniform(kh[1], (B, EMB), minval=-a, maxval=a, dtype=F32)

    respf = response.astype(F32).T[:T].reshape(T, 1, B)
    bsumT = (bih + bhh).reshape(4 * EMB, 1)
    outp = _passR(g2[0], g2[1], g2[2], respf, Wih[:EMB].T, Wih[EMB:].T,
                  Whh.T, bsumT, q_W.T, q_b.reshape(EMB, 1), k_W.T,
                  k_b.reshape(EMB, 1), w_W[:EMB], w_W[EMB:], emb_r,
                  h0.T, c0.T)

    res = outp.reshape(T, B).T
    return jnp.concatenate([jnp.zeros((B, 1), F32), res], axis=1)


# speedup vs baseline: 26.9513x; 1.0690x over previous
import functools

import jax
import jax.numpy as jnp
import numpy as np
from jax import lax
from jax.experimental import pallas as pl
from jax.experimental.pallas import tpu as pltpu
from jax.experimental.pallas import tpu_sc as plsc

NUM_Q = 20000
NUM_S = 500
EMB = 128
B = 1024
S = 20
RANK_K = 10
T = S - 1
BQ = 512
NSP = 512
GRID_Q = (NUM_Q + BQ - 1) // BQ
NC, NS_SC = 2, 16
NW = NC * NS_SC
HI = lax.Precision.HIGHEST
F32 = jnp.float32


def _dot(a, b):
    return jnp.dot(a, b, preferred_element_type=F32)


def _dot_bt(a, b):
    return lax.dot_general(a, b, (((1,), (1,)), ((), ())),
                           preferred_element_type=F32)


def _sc_gather(table, idx, rows, chunk):
    per_w = rows // NW
    n_chunks = per_w // chunk
    nb = min(3, n_chunks)
    mesh = plsc.VectorSubcoreMesh(core_axis_name="c", subcore_axis_name="s")

    @functools.partial(
        pl.kernel,
        out_type=jax.ShapeDtypeStruct((rows, EMB), F32),
        mesh=mesh,
        scratch_types=(
            [pltpu.VMEM((chunk,), jnp.int32) for _ in range(nb)]
            + [pltpu.VMEM((chunk, EMB), F32) for _ in range(nb)]
            + [pltpu.SemaphoreType.DMA, pltpu.SemaphoreType.DMA]
        ),
    )
    def gather(table_hbm, idx_hbm, out_hbm, *rest):
        ibufs = rest[:nb]
        bufs = rest[nb:2 * nb]
        gsem, wsem = rest[2 * nb], rest[2 * nb + 1]
        wid = lax.axis_index("s") * NC + lax.axis_index("c")
        base = wid * per_w

        def start(k):
            pltpu.sync_copy(idx_hbm.at[pl.ds(base + k * chunk, chunk)],
                            ibufs[k % nb])
            return pltpu.async_copy(table_hbm.at[ibufs[k % nb]],
                                    bufs[k % nb], gsem)

        gps = {0: start(0)}
        wps = {}
        for k in range(n_chunks):
            if k + 1 < n_chunks:
                if k + 1 >= nb:
                    wps[k + 1 - nb].wait()
                gps[k + 1] = start(k + 1)
            gps[k].wait()
            wps[k] = pltpu.async_copy(
                bufs[k % nb], out_hbm.at[pl.ds(base + k * chunk, chunk)], wsem)
        for k in range(max(0, n_chunks - nb), n_chunks):
            wps[k].wait()

    return gather(table, idx)


def _passA_body(qn_ref, eq_ref, es_ref, W2_ref, b2_ref, W0_ref, b0_ref,
                tall_ref, t0a_ref):
    qn = qn_ref[...]
    iot = lax.broadcasted_iota(jnp.int32, (BQ, NSP), 1)
    counts = jnp.zeros((BQ, NSP), jnp.bfloat16)
    for j in range(4):
        counts += (qn[:, j:j + 1] == iot).astype(jnp.bfloat16)
    qmean = _dot(counts, es_ref[...].astype(jnp.bfloat16)) * 0.25
    x = eq_ref[...] + qmean
    tall_ref[0] = eq_ref[...]
    tall_ref[1] = jnp.tanh(_dot(x, W2_ref[...]) + b2_ref[...])
    t0a_ref[...] = jnp.tanh(_dot(x, W0_ref[...]) + b0_ref[...])


def _passA(qn, emb_q, es512, W2, b2, W0, b0):
    return pl.pallas_call(
        _passA_body,
        grid=(GRID_Q,),
        in_specs=[
            pl.BlockSpec((BQ, 4), lambda i: (i, 0)),
            pl.BlockSpec((BQ, EMB), lambda i: (i, 0)),
            pl.BlockSpec((NSP, EMB), lambda i: (0, 0)),
            pl.BlockSpec((EMB, EMB), lambda i: (0, 0)),
            pl.BlockSpec((1, EMB), lambda i: (0, 0)),
            pl.BlockSpec((EMB, EMB), lambda i: (0, 0)),
            pl.BlockSpec((1, EMB), lambda i: (0, 0)),
        ],
        out_specs=[
            pl.BlockSpec((2, BQ, EMB), lambda i: (0, i, 0)),
            pl.BlockSpec((BQ, EMB), lambda i: (i, 0)),
        ],
        out_shape=[
            jax.ShapeDtypeStruct((2, NUM_Q, EMB), F32),
            jax.ShapeDtypeStruct((NUM_Q, EMB), F32),
        ],
    )(qn, emb_q, es512, W2, b2, W0, b0)


def _passB_body(es_ref, g_ref, W1_ref, b1_ref, t1a_ref, t1b_ref):
    g = g_ref[...]
    sm0 = (g[0] + g[1] + g[2] + g[3]) * 0.25
    t1a = jnp.tanh(_dot(es_ref[...] + sm0, W1_ref[...]) + b1_ref[...])
    sm1 = (g[4] + g[5] + g[6] + g[7]) * 0.25
    t1b = jnp.tanh(_dot(t1a + sm1, W1_ref[...]) + b1_ref[...])
    t1a_ref[...] = t1a
    t1b_ref[...] = t1b


def _passB(es512, g, W1, b1):
    return pl.pallas_call(
        _passB_body,
        out_shape=[
            jax.ShapeDtypeStruct((NSP, EMB), F32),
            jax.ShapeDtypeStruct((NSP, EMB), F32),
        ],
    )(es512, g, W1, b1)


def _passC_body(qn_ref, eq_ref, t0a_ref, qs_ref, t1a_ref, t1b_ref, es_ref,
                W0_ref, b0_ref, lw_ref, lb_ref, fw_ref, fb_ref, tall_ref):
    qn = qn_ref[...]
    iot = lax.broadcasted_iota(jnp.int32, (BQ, NSP), 1)
    counts = jnp.zeros((BQ, NSP), jnp.bfloat16)
    for j in range(4):
        counts += (qn[:, j:j + 1] == iot).astype(jnp.bfloat16)
    qm1 = _dot(counts, t1a_ref[...].astype(jnp.bfloat16)) * 0.25
    t0b = jnp.tanh(_dot(t0a_ref[...] + qm1, W0_ref[...]) + b0_ref[...])
    qm2 = _dot(counts, t1b_ref[...].astype(jnp.bfloat16)) * 0.25
    t0c = jnp.tanh(_dot(t0b + qm2, W0_ref[...]) + b0_ref[...])
    qfin = jnp.tanh(_dot(t0c, lw_ref[...]) + lb_ref[...])
    tall_ref[0] = jnp.maximum(_dot(eq_ref[...], fw_ref[...]) + fb_ref[...], 0.0)
    tall_ref[1] = jnp.maximum(_dot(qfin, fw_ref[...]) + fb_ref[...], 0.0)
    tall_ref[2] = eq_ref[...]
    qs = qs_ref[...]
    esum = _dot(qs.astype(jnp.bfloat16), es_ref[...].astype(jnp.bfloat16))
    rs = jnp.sum(qs, axis=1, keepdims=True)
    tall_ref[3] = esum / jnp.maximum(rs, 1.0)


def _passC(qn, emb_q, t0a, qs_table, t1a, t1b, emb_s, W0, b0, lw, lb, fw, fb):
    return pl.pallas_call(
        _passC_body,
        grid=(GRID_Q,),
        in_specs=[
            pl.BlockSpec((BQ, 4), lambda i: (i, 0)),
            pl.BlockSpec((BQ, EMB), lambda i: (i, 0)),
            pl.BlockSpec((BQ, EMB), lambda i: (i, 0)),
            pl.BlockSpec((BQ, NUM_S), lambda i: (i, 0)),
            pl.BlockSpec((NSP, EMB), lambda i: (0, 0)),
            pl.BlockSpec((NSP, EMB), lambda i: (0, 0)),
            pl.BlockSpec((NUM_S, EMB), lambda i: (0, 0)),
            pl.BlockSpec((EMB, EMB), lambda i: (0, 0)),
            pl.BlockSpec((1, EMB), lambda i: (0, 0)),
            pl.BlockSpec((EMB, EMB), lambda i: (0, 0)),
            pl.BlockSpec((1, EMB), lambda i: (0, 0)),
            pl.BlockSpec((EMB, EMB), lambda i: (0, 0)),
            pl.BlockSpec((1, EMB), lambda i: (0, 0)),
        ],
        out_specs=pl.BlockSpec((4, BQ, EMB), lambda i: (0, i, 0)),
        out_shape=jax.ShapeDtypeStruct((4, NUM_Q, EMB), F32),
    )(qn, emb_q, t0a, qs_table, t1a, t1b, emb_s, W0, b0, lw, lb, fw, fb)


def _passR_body(xsel_ref, eqn_ref, esk_ref, resp_ref, WihAT_ref, WihBT_ref,
                WhhT_ref, bsumT_ref, qWT_ref, qbT_ref, kWT_ref, kbT_ref,
                w1_ref, w2_ref, er_ref, h0T_ref, c0T_ref, out_ref,
                hT_s, cT_s, qring, lqring):
    t = pl.program_id(0)

    @pl.when(t == 0)
    def _init():
        hT_s[...] = h0T_ref[...]
        cT_s[...] = c0T_ref[...]

    rWT = _dot_bt(WihBT_ref[...], er_ref[...])
    resp = resp_ref[0]
    gatesT = (_dot_bt(WihAT_ref[...], xsel_ref[0, 0])
              + _dot(WhhT_ref[...], hT_s[...]) + bsumT_ref[...]
              + rWT[:, 0:1] + resp * (rWT[:, 1:2] - rWT[:, 0:1]))
    ig = jax.nn.sigmoid(gatesT[0:EMB])
    fg = jax.nn.sigmoid(gatesT[EMB:2 * EMB])
    gg = jnp.tanh(gatesT[2 * EMB:3 * EMB])
    og = jax.nn.sigmoid(gatesT[3 * EMB:4 * EMB])
    cT = fg * cT_s[...] + ig * gg
    hT = og * jnp.tanh(cT)
    cT_s[...] = cT
    hT_s[...] = hT

    qhT = _dot(qWT_ref[...], hT) + qbT_ref[...]
    lq_t = jnp.sum(qhT * w1_ref[...], axis=0, keepdims=True)
    slot = lax.rem(t, RANK_K)
    for s_i in range(RANK_K):
        @pl.when(slot == s_i)
        def _store(s_i=s_i):
            qring[s_i] = qhT
            lqring[s_i] = lq_t

    KmT0 = _dot_bt(kWT_ref[...], eqn_ref[0, 0]) + kbT_ref[...]
    KmT1 = _dot_bt(kWT_ref[...], esk_ref[0, 0]) + kbT_ref[...]
    w2 = w2_ref[...]
    lk0 = jnp.sum(KmT0 * w2, axis=0, keepdims=True)
    lk1 = jnp.sum(KmT1 * w2, axis=0, keepdims=True)

    ls, gs = [], []
    mx = jnp.full((1, B), -1e30, F32)
    for s_i in range(RANK_K):
        valid = jnp.logical_or(s_i <= t, t >= RANK_K)
        qrowT = qring[s_i]
        lq_s = lqring[s_i]
        for km, lk in ((KmT0, lk0), (KmT1, lk1)):
            l = jnp.where(valid, lq_s + lk, -1e30)
            g = jnp.where(
                valid,
                jax.nn.sigmoid(jnp.sum(qrowT * km, axis=0, keepdims=True)),
                0.0)
            ls.append(l)
            gs.append(g)
            mx = jnp.maximum(mx, l)
    num = jnp.zeros((1, B), F32)
    den = jnp.zeros((1, B), F32)
    for l, g in zip(ls, gs):
        e = jnp.exp(l - mx)
        num += e * g
        den += e
    out_ref[0] = num / den


def _passR(xsel, eqn, esk, respf, WihAT, WihBT, WhhT, bsumT, qWT, qbT,
           kWT, kbT, w1, w2, emb_r, h0T, c0T):
    full = lambda shape: pl.BlockSpec(shape, lambda t: tuple(0 for _ in shape))
    return pl.pallas_call(
        _passR_body,
        grid=(T,),
        in_specs=[
            pl.BlockSpec((1, 1, B, EMB), lambda t: (0, t, 0, 0)),
            pl.BlockSpec((1, 1, B, EMB), lambda t: (1, t, 0, 0)),
            pl.BlockSpec((1, 1, B, EMB), lambda t: (2, t, 0, 0)),
            pl.BlockSpec((1, 1, B), lambda t: (t, 0, 0)),
            full((4 * EMB, EMB)),
            full((4 * EMB, EMB)),
            full((4 * EMB, EMB)),
            full((4 * EMB, 1)),
            full((EMB, EMB)),
            full((EMB, 1)),
            full((EMB, EMB)),
            full((EMB, 1)),
            full((EMB, 1)),
            full((EMB, 1)),
            full((2, EMB)),
            full((EMB, B)),
            full((EMB, B)),
        ],
        out_specs=pl.BlockSpec((1, 1, B), lambda t: (t, 0, 0)),
        out_shape=jax.ShapeDtypeStruct((T, 1, B), F32),
        scratch_shapes=[
            pltpu.VMEM((EMB, B), F32),
            pltpu.VMEM((EMB, B), F32),
            pltpu.VMEM((RANK_K, EMB, B), F32),
            pltpu.VMEM((RANK_K, 1, B), F32),
        ],
    )(xsel, eqn, esk, respf, WihAT, WihBT, WhhT, bsumT, qWT, qbT, kWT, kbT,
      w1, w2, emb_r, h0T, c0T)


def kernel(question, response, mask, q_neighbors, s_neighbors, qs_table,
           emb_q, emb_s, emb_r, ft_W, ft_b, agg_W, agg_b, last_W, last_b,
           Wih, Whh, bih, bhh, q_W, q_b, k_W, k_b, w_W, w_b):
    q = question.astype(jnp.int32)
    msk = mask.astype(jnp.int32)
    qn = q_neighbors.astype(jnp.int32)
    sn = s_neighbors.astype(jnp.int32)
    W0, W1, W2 = agg_W[0], agg_W[1], agg_W[2]
    b0 = agg_b[0].reshape(1, EMB)
    b1 = agg_b[1].reshape(1, EMB)
    b2 = agg_b[2].reshape(1, EMB)
    lb = last_b.reshape(1, EMB)
    fb = ft_b.reshape(1, EMB)
    es512 = jnp.pad(emb_s, ((0, NSP - NUM_S), (0, 0)))

    tall_a, t0a = _passA(qn, emb_q, es512, W2, b2, W0, b0)

    snp = jnp.pad(sn, ((0, NSP - NUM_S), (0, 0))).T.reshape(-1)
    idx1 = jnp.concatenate([snp, snp + NUM_Q])
    g1 = _sc_gather(tall_a.reshape(2 * NUM_Q, EMB), idx1, 4096, 128)

    t1a, t1b = _passB(es512, g1.reshape(8, NSP, EMB), W1, b1)

    tall = _passC(qn, emb_q, t0a, qs_table, t1a, t1b, emb_s,
                  W0, b0, last_W, lb, ft_W, fb)

    qT = q.T
    sel = (qT[:T] + NUM_Q * msk.T[:T]).reshape(-1)
    nxt = qT[1:].reshape(-1)
    idx2 = jnp.concatenate([sel, 2 * NUM_Q + nxt, 3 * NUM_Q + nxt])
    g2 = _sc_gather(tall.reshape(4 * NUM_Q, EMB), idx2, 3 * T * B, 96)
    g2 = g2.reshape(3, T, B, EMB)

    a = float(np.sqrt(6.0 / (B + EMB)))
    kh = jax.random.split(jax.random.key(42))
    h0 = jax.random.uniform(kh[0], (B, EMB), minval=-a, maxval=a, dtype=F32)
    c0 = jax.random.uniform(kh[1], (B, EMB), minval=-a, maxval=a, dtype=F32)

    respf = response.astype(F32).T[:T].reshape(T, 1, B)
    bsumT = (bih + bhh).reshape(4 * EMB, 1)
    outp = _passR(g2, g2, g2, respf, Wih[:EMB].T, Wih[EMB:].T,
                  Whh.T, bsumT, q_W.T, q_b.reshape(EMB, 1), k_W.T,
                  k_b.reshape(EMB, 1), w_W[:EMB], w_W[EMB:], emb_r,
                  h0.T, c0.T)

    res = outp.reshape(T, B).T
    return jnp.concatenate([jnp.zeros((B, 1), F32), res], axis=1)


# speedup vs baseline: 27.4320x; 1.0178x over previous
import functools

import jax
import jax.numpy as jnp
import numpy as np
from jax import lax
from jax.experimental import pallas as pl
from jax.experimental.pallas import tpu as pltpu
from jax.experimental.pallas import tpu_sc as plsc

NUM_Q = 20000
NUM_S = 500
EMB = 128
B = 1024
S = 20
RANK_K = 10
T = S - 1
BQ = 512
NSP = 512
GRID_Q = (NUM_Q + BQ - 1) // BQ
NC, NS_SC = 2, 16
NW = NC * NS_SC
HI = lax.Precision.HIGHEST
F32 = jnp.float32


def _dot(a, b):
    return jnp.dot(a, b, preferred_element_type=F32)


def _dot_bt(a, b):
    return lax.dot_general(a, b, (((1,), (1,)), ((), ())),
                           preferred_element_type=F32)


def _sc_gather(table, idx, rows, chunk):
    per_w = rows // NW
    n_chunks = per_w // chunk
    nb = min(3, n_chunks)
    mesh = plsc.VectorSubcoreMesh(core_axis_name="c", subcore_axis_name="s")

    @functools.partial(
        pl.kernel,
        out_type=jax.ShapeDtypeStruct((rows, EMB), F32),
        mesh=mesh,
        scratch_types=(
            [pltpu.VMEM((chunk,), jnp.int32) for _ in range(nb)]
            + [pltpu.VMEM((chunk, EMB), F32) for _ in range(nb)]
            + [pltpu.SemaphoreType.DMA, pltpu.SemaphoreType.DMA]
        ),
    )
    def gather(table_hbm, idx_hbm, out_hbm, *rest):
        ibufs = rest[:nb]
        bufs = rest[nb:2 * nb]
        gsem, wsem = rest[2 * nb], rest[2 * nb + 1]
        wid = lax.axis_index("s") * NC + lax.axis_index("c")
        base = wid * per_w

        def start(k):
            pltpu.sync_copy(idx_hbm.at[pl.ds(base + k * chunk, chunk)],
                            ibufs[k % nb])
            return pltpu.async_copy(table_hbm.at[ibufs[k % nb]],
                                    bufs[k % nb], gsem)

        gps = {0: start(0)}
        wps = {}
        for k in range(n_chunks):
            if k + 1 < n_chunks:
                if k + 1 >= nb:
                    wps[k + 1 - nb].wait()
                gps[k + 1] = start(k + 1)
            gps[k].wait()
            wps[k] = pltpu.async_copy(
                bufs[k % nb], out_hbm.at[pl.ds(base + k * chunk, chunk)], wsem)
        for k in range(max(0, n_chunks - nb), n_chunks):
            wps[k].wait()

    return gather(table, idx)


def _passA_body(qn_ref, eq_ref, es_ref, W2_ref, b2_ref, W0_ref, b0_ref,
                tall_ref, t0a_ref):
    qn = qn_ref[...]
    iot = lax.broadcasted_iota(jnp.int32, (BQ, NSP), 1)
    counts = jnp.zeros((BQ, NSP), jnp.bfloat16)
    for j in range(4):
        counts += (qn[:, j:j + 1] == iot).astype(jnp.bfloat16)
    qmean = _dot(counts, es_ref[...].astype(jnp.bfloat16)) * 0.25
    x = eq_ref[...] + qmean
    tall_ref[0] = eq_ref[...]
    tall_ref[1] = jnp.tanh(_dot(x, W2_ref[...]) + b2_ref[...])
    t0a_ref[...] = jnp.tanh(_dot(x, W0_ref[...]) + b0_ref[...])


def _passA(qn, emb_q, es512, W2, b2, W0, b0):
    return pl.pallas_call(
        _passA_body,
        grid=(GRID_Q,),
        in_specs=[
            pl.BlockSpec((BQ, 4), lambda i: (i, 0)),
            pl.BlockSpec((BQ, EMB), lambda i: (i, 0)),
            pl.BlockSpec((NSP, EMB), lambda i: (0, 0)),
            pl.BlockSpec((EMB, EMB), lambda i: (0, 0)),
            pl.BlockSpec((1, EMB), lambda i: (0, 0)),
            pl.BlockSpec((EMB, EMB), lambda i: (0, 0)),
            pl.BlockSpec((1, EMB), lambda i: (0, 0)),
        ],
        out_specs=[
            pl.BlockSpec((2, BQ, EMB), lambda i: (0, i, 0)),
            pl.BlockSpec((BQ, EMB), lambda i: (i, 0)),
        ],
        out_shape=[
            jax.ShapeDtypeStruct((2, NUM_Q, EMB), F32),
            jax.ShapeDtypeStruct((NUM_Q, EMB), F32),
        ],
    )(qn, emb_q, es512, W2, b2, W0, b0)


def _passB_body(es_ref, g_ref, W1_ref, b1_ref, t1a_ref, t1b_ref):
    g = g_ref[...]
    sm0 = (g[0] + g[1] + g[2] + g[3]) * 0.25
    t1a = jnp.tanh(_dot(es_ref[...] + sm0, W1_ref[...]) + b1_ref[...])
    sm1 = (g[4] + g[5] + g[6] + g[7]) * 0.25
    t1b = jnp.tanh(_dot(t1a + sm1, W1_ref[...]) + b1_ref[...])
    t1a_ref[...] = t1a
    t1b_ref[...] = t1b


def _passB(es512, g, W1, b1):
    return pl.pallas_call(
        _passB_body,
        out_shape=[
            jax.ShapeDtypeStruct((NSP, EMB), F32),
            jax.ShapeDtypeStruct((NSP, EMB), F32),
        ],
    )(es512, g, W1, b1)


def _passC_body(qn_ref, eq_ref, t0a_ref, qs_ref, t1a_ref, t1b_ref, es_ref,
                W0_ref, b0_ref, lw_ref, lb_ref, fw_ref, fb_ref, tall_ref):
    qn = qn_ref[...]
    iot = lax.broadcasted_iota(jnp.int32, (BQ, NSP), 1)
    counts = jnp.zeros((BQ, NSP), jnp.bfloat16)
    for j in range(4):
        counts += (qn[:, j:j + 1] == iot).astype(jnp.bfloat16)
    qm1 = _dot(counts, t1a_ref[...].astype(jnp.bfloat16)) * 0.25
    t0b = jnp.tanh(_dot(t0a_ref[...] + qm1, W0_ref[...]) + b0_ref[...])
    qm2 = _dot(counts, t1b_ref[...].astype(jnp.bfloat16)) * 0.25
    t0c = jnp.tanh(_dot(t0b + qm2, W0_ref[...]) + b0_ref[...])
    qfin = jnp.tanh(_dot(t0c, lw_ref[...]) + lb_ref[...])
    tall_ref[0] = jnp.maximum(_dot(eq_ref[...], fw_ref[...]) + fb_ref[...], 0.0)
    tall_ref[1] = jnp.maximum(_dot(qfin, fw_ref[...]) + fb_ref[...], 0.0)
    tall_ref[2] = eq_ref[...]
    qs = qs_ref[...]
    esum = _dot(qs.astype(jnp.bfloat16), es_ref[...].astype(jnp.bfloat16))
    rs = jnp.sum(qs, axis=1, keepdims=True)
    tall_ref[3] = esum / jnp.maximum(rs, 1.0)


def _passC(qn, emb_q, t0a, qs_table, t1a, t1b, emb_s, W0, b0, lw, lb, fw, fb):
    return pl.pallas_call(
        _passC_body,
        grid=(GRID_Q,),
        in_specs=[
            pl.BlockSpec((BQ, 4), lambda i: (i, 0)),
            pl.BlockSpec((BQ, EMB), lambda i: (i, 0)),
            pl.BlockSpec((BQ, EMB), lambda i: (i, 0)),
            pl.BlockSpec((BQ, NUM_S), lambda i: (i, 0)),
            pl.BlockSpec((NSP, EMB), lambda i: (0, 0)),
            pl.BlockSpec((NSP, EMB), lambda i: (0, 0)),
            pl.BlockSpec((NUM_S, EMB), lambda i: (0, 0)),
            pl.BlockSpec((EMB, EMB), lambda i: (0, 0)),
            pl.BlockSpec((1, EMB), lambda i: (0, 0)),
            pl.BlockSpec((EMB, EMB), lambda i: (0, 0)),
            pl.BlockSpec((1, EMB), lambda i: (0, 0)),
            pl.BlockSpec((EMB, EMB), lambda i: (0, 0)),
            pl.BlockSpec((1, EMB), lambda i: (0, 0)),
        ],
        out_specs=pl.BlockSpec((4, BQ, EMB), lambda i: (0, i, 0)),
        out_shape=jax.ShapeDtypeStruct((4, NUM_Q, EMB), F32),
    )(qn, emb_q, t0a, qs_table, t1a, t1b, emb_s, W0, b0, lw, lb, fw, fb)


def _passR_body(xsel_ref, eqn_ref, esk_ref, resp_ref, WihAT_ref, WihBT_ref,
                WhhT_ref, bsumT_ref, qWT_ref, qbT_ref, kWT_ref, kbT_ref,
                w1_ref, w2_ref, er_ref, h0T_ref, c0T_ref, out_ref,
                hT_s, cT_s, qring, lqring):
    t = pl.program_id(0)

    @pl.when(t == 0)
    def _init():
        hT_s[...] = h0T_ref[...]
        cT_s[...] = c0T_ref[...]

    rWT = _dot_bt(WihBT_ref[...], er_ref[...])
    resp = resp_ref[0]
    gatesT = (_dot_bt(WihAT_ref[...], xsel_ref[0, 0])
              + _dot(WhhT_ref[...], hT_s[...]) + bsumT_ref[...]
              + rWT[:, 0:1] + resp * (rWT[:, 1:2] - rWT[:, 0:1]))
    ig = jax.nn.sigmoid(gatesT[0:EMB])
    fg = jax.nn.sigmoid(gatesT[EMB:2 * EMB])
    gg = jnp.tanh(gatesT[2 * EMB:3 * EMB])
    og = jax.nn.sigmoid(gatesT[3 * EMB:4 * EMB])
    cT = fg * cT_s[...] + ig * gg
    hT = og * jnp.tanh(cT)
    cT_s[...] = cT
    hT_s[...] = hT

    qhT = _dot(qWT_ref[...], hT) + qbT_ref[...]
    lq_t = jnp.sum(qhT * w1_ref[...], axis=0, keepdims=True)
    slot = lax.rem(t, RANK_K)
    qring[pl.ds(slot, 1)] = qhT[None]
    lqring[pl.ds(slot, 1)] = lq_t[None]

    KmT0 = _dot_bt(kWT_ref[...], eqn_ref[0, 0]) + kbT_ref[...]
    KmT1 = _dot_bt(kWT_ref[...], esk_ref[0, 0]) + kbT_ref[...]
    w2 = w2_ref[...]
    lk0 = jnp.sum(KmT0 * w2, axis=0, keepdims=True)
    lk1 = jnp.sum(KmT1 * w2, axis=0, keepdims=True)

    ls, gs = [], []
    mx = jnp.full((1, B), -1e30, F32)
    for s_i in range(RANK_K):
        valid = jnp.logical_or(s_i <= t, t >= RANK_K)
        qrowT = qring[s_i]
        lq_s = lqring[s_i]
        for km, lk in ((KmT0, lk0), (KmT1, lk1)):
            l = jnp.where(valid, lq_s + lk, -1e30)
            g = jnp.where(
                valid,
                jax.nn.sigmoid(jnp.sum(qrowT * km, axis=0, keepdims=True)),
                0.0)
            ls.append(l)
            gs.append(g)
            mx = jnp.maximum(mx, l)
    num = jnp.zeros((1, B), F32)
    den = jnp.zeros((1, B), F32)
    for l, g in zip(ls, gs):
        e = jnp.exp(l - mx)
        num += e * g
        den += e
    out_ref[0] = num / den


def _passR(xsel, eqn, esk, respf, WihAT, WihBT, WhhT, bsumT, qWT, qbT,
           kWT, kbT, w1, w2, emb_r, h0T, c0T):
    full = lambda shape: pl.BlockSpec(shape, lambda t: tuple(0 for _ in shape))
    return pl.pallas_call(
        _passR_body,
        grid=(T,),
        in_specs=[
            pl.BlockSpec((1, 1, B, EMB), lambda t: (0, t, 0, 0)),
            pl.BlockSpec((1, 1, B, EMB), lambda t: (1, t, 0, 0)),
            pl.BlockSpec((1, 1, B, EMB), lambda t: (2, t, 0, 0)),
            pl.BlockSpec((1, 1, B), lambda t: (t, 0, 0)),
            full((4 * EMB, EMB)),
            full((4 * EMB, EMB)),
            full((4 * EMB, EMB)),
            full((4 * EMB, 1)),
            full((EMB, EMB)),
            full((EMB, 1)),
            full((EMB, EMB)),
            full((EMB, 1)),
            full((EMB, 1)),
            full((EMB, 1)),
            full((2, EMB)),
            full((EMB, B)),
            full((EMB, B)),
        ],
        out_specs=pl.BlockSpec((1, 1, B), lambda t: (t, 0, 0)),
        out_shape=jax.ShapeDtypeStruct((T, 1, B), F32),
        scratch_shapes=[
            pltpu.VMEM((EMB, B), F32),
            pltpu.VMEM((EMB, B), F32),
            pltpu.VMEM((RANK_K, EMB, B), F32),
            pltpu.VMEM((RANK_K, 1, B), F32),
        ],
    )(xsel, eqn, esk, respf, WihAT, WihBT, WhhT, bsumT, qWT, qbT, kWT, kbT,
      w1, w2, emb_r, h0T, c0T)


def kernel(question, response, mask, q_neighbors, s_neighbors, qs_table,
           emb_q, emb_s, emb_r, ft_W, ft_b, agg_W, agg_b, last_W, last_b,
           Wih, Whh, bih, bhh, q_W, q_b, k_W, k_b, w_W, w_b):
    q = question.astype(jnp.int32)
    msk = mask.astype(jnp.int32)
    qn = q_neighbors.astype(jnp.int32)
    sn = s_neighbors.astype(jnp.int32)
    W0, W1, W2 = agg_W[0], agg_W[1], agg_W[2]
    b0 = agg_b[0].reshape(1, EMB)
    b1 = agg_b[1].reshape(1, EMB)
    b2 = agg_b[2].reshape(1, EMB)
    lb = last_b.reshape(1, EMB)
    fb = ft_b.reshape(1, EMB)
    es512 = jnp.pad(emb_s, ((0, NSP - NUM_S), (0, 0)))

    tall_a, t0a = _passA(qn, emb_q, es512, W2, b2, W0, b0)

    snp = jnp.pad(sn, ((0, NSP - NUM_S), (0, 0))).T.reshape(-1)
    idx1 = jnp.concatenate([snp, snp + NUM_Q])
    g1 = _sc_gather(tall_a.reshape(2 * NUM_Q, EMB), idx1, 4096, 128)

    t1a, t1b = _passB(es512, g1.reshape(8, NSP, EMB), W1, b1)

    tall = _passC(qn, emb_q, t0a, qs_table, t1a, t1b, emb_s,
                  W0, b0, last_W, lb, ft_W, fb)

    qT = q.T
    sel = (qT[:T] + NUM_Q * msk.T[:T]).reshape(-1)
    nxt = qT[1:].reshape(-1)
    idx2 = jnp.concatenate([sel, 2 * NUM_Q + nxt, 3 * NUM_Q + nxt])
    g2 = _sc_gather(tall.reshape(4 * NUM_Q, EMB), idx2, 3 * T * B, 96)
    g2 = g2.reshape(3, T, B, EMB)

    a = float(np.sqrt(6.0 / (B + EMB)))
    kh = jax.random.split(jax.random.key(42))
    h0 = jax.random.uniform(kh[0], (B, EMB), minval=-a, maxval=a, dtype=F32)
    c0 = jax.random.uniform(kh[1], (B, EMB), minval=-a, maxval=a, dtype=F32)

    respf = response.astype(F32).T[:T].reshape(T, 1, B)
    bsumT = (bih + bhh).reshape(4 * EMB, 1)
    outp = _passR(g2, g2, g2, respf, Wih[:EMB].T, Wih[EMB:].T,
                  Whh.T, bsumT, q_W.T, q_b.reshape(EMB, 1), k_W.T,
                  k_b.reshape(EMB, 1), w_W[:EMB], w_W[EMB:], emb_r,
                  h0.T, c0.T)

    res = outp.reshape(T, B).T
    return jnp.concatenate([jnp.zeros((B, 1), F32), res], axis=1)


# speedup vs baseline: 30.6679x; 1.1180x over previous
import functools

import jax
import jax.numpy as jnp
import numpy as np
from jax import lax
from jax.experimental import pallas as pl
from jax.experimental.pallas import tpu as pltpu
from jax.experimental.pallas import tpu_sc as plsc

NUM_Q = 20000
NUM_S = 500
EMB = 128
B = 1024
S = 20
RANK_K = 10
T = S - 1
BQ = 1024
NSP = 512
GRID_Q = (NUM_Q + BQ - 1) // BQ
NC, NS_SC = 2, 16
NW = NC * NS_SC
HI = lax.Precision.HIGHEST
F32 = jnp.float32


def _dot(a, b):
    return jnp.dot(a, b, preferred_element_type=F32)


def _dot_bt(a, b):
    return lax.dot_general(a, b, (((1,), (1,)), ((), ())),
                           preferred_element_type=F32)


def _sc_gather(table, idx, rows, chunk):
    per_w = rows // NW
    n_chunks = per_w // chunk
    nb = min(3, n_chunks)
    mesh = plsc.VectorSubcoreMesh(core_axis_name="c", subcore_axis_name="s")

    @functools.partial(
        pl.kernel,
        out_type=jax.ShapeDtypeStruct((rows, EMB), F32),
        mesh=mesh,
        scratch_types=(
            [pltpu.VMEM((chunk,), jnp.int32) for _ in range(nb)]
            + [pltpu.VMEM((chunk, EMB), F32) for _ in range(nb)]
            + [pltpu.SemaphoreType.DMA, pltpu.SemaphoreType.DMA]
        ),
    )
    def gather(table_hbm, idx_hbm, out_hbm, *rest):
        ibufs = rest[:nb]
        bufs = rest[nb:2 * nb]
        gsem, wsem = rest[2 * nb], rest[2 * nb + 1]
        wid = lax.axis_index("s") * NC + lax.axis_index("c")
        base = wid * per_w

        def start(k):
            pltpu.sync_copy(idx_hbm.at[pl.ds(base + k * chunk, chunk)],
                            ibufs[k % nb])
            return pltpu.async_copy(table_hbm.at[ibufs[k % nb]],
                                    bufs[k % nb], gsem)

        gps = {0: start(0)}
        wps = {}
        for k in range(n_chunks):
            if k + 1 < n_chunks:
                if k + 1 >= nb:
                    wps[k + 1 - nb].wait()
                gps[k + 1] = start(k + 1)
            gps[k].wait()
            wps[k] = pltpu.async_copy(
                bufs[k % nb], out_hbm.at[pl.ds(base + k * chunk, chunk)], wsem)
        for k in range(max(0, n_chunks - nb), n_chunks):
            wps[k].wait()

    return gather(table, idx)


def _passA_body(qn_ref, eq_ref, es_ref, W2_ref, b2_ref, W0_ref, b0_ref,
                tall_ref, t0a_ref):
    qn = qn_ref[...]
    iot = lax.broadcasted_iota(jnp.int32, (BQ, NSP), 1)
    counts = jnp.zeros((BQ, NSP), jnp.bfloat16)
    for j in range(4):
        counts += (qn[:, j:j + 1] == iot).astype(jnp.bfloat16)
    qmean = _dot(counts, es_ref[...].astype(jnp.bfloat16)) * 0.25
    x = eq_ref[...] + qmean
    tall_ref[0] = eq_ref[...]
    tall_ref[1] = jnp.tanh(_dot(x, W2_ref[...]) + b2_ref[...])
    t0a_ref[...] = jnp.tanh(_dot(x, W0_ref[...]) + b0_ref[...])


def _passA(qn, emb_q, es512, W2, b2, W0, b0):
    return pl.pallas_call(
        _passA_body,
        grid=(GRID_Q,),
        in_specs=[
            pl.BlockSpec((BQ, 4), lambda i: (i, 0)),
            pl.BlockSpec((BQ, EMB), lambda i: (i, 0)),
            pl.BlockSpec((NSP, EMB), lambda i: (0, 0)),
            pl.BlockSpec((EMB, EMB), lambda i: (0, 0)),
            pl.BlockSpec((1, EMB), lambda i: (0, 0)),
            pl.BlockSpec((EMB, EMB), lambda i: (0, 0)),
            pl.BlockSpec((1, EMB), lambda i: (0, 0)),
        ],
        out_specs=[
            pl.BlockSpec((2, BQ, EMB), lambda i: (0, i, 0)),
            pl.BlockSpec((BQ, EMB), lambda i: (i, 0)),
        ],
        out_shape=[
            jax.ShapeDtypeStruct((2, NUM_Q, EMB), F32),
            jax.ShapeDtypeStruct((NUM_Q, EMB), F32),
        ],
    )(qn, emb_q, es512, W2, b2, W0, b0)


def _passB_body(es_ref, g_ref, W1_ref, b1_ref, t1a_ref, t1b_ref):
    g = g_ref[...]
    sm0 = (g[0] + g[1] + g[2] + g[3]) * 0.25
    t1a = jnp.tanh(_dot(es_ref[...] + sm0, W1_ref[...]) + b1_ref[...])
    sm1 = (g[4] + g[5] + g[6] + g[7]) * 0.25
    t1b = jnp.tanh(_dot(t1a + sm1, W1_ref[...]) + b1_ref[...])
    t1a_ref[...] = t1a
    t1b_ref[...] = t1b


def _passB(es512, g, W1, b1):
    return pl.pallas_call(
        _passB_body,
        out_shape=[
            jax.ShapeDtypeStruct((NSP, EMB), F32),
            jax.ShapeDtypeStruct((NSP, EMB), F32),
        ],
    )(es512, g, W1, b1)


def _passC_body(qn_ref, eq_ref, t0a_ref, qs_ref, t1a_ref, t1b_ref, es_ref,
                W0_ref, b0_ref, lw_ref, lb_ref, fw_ref, fb_ref, tall_ref):
    qn = qn_ref[...]
    iot = lax.broadcasted_iota(jnp.int32, (BQ, NSP), 1)
    counts = jnp.zeros((BQ, NSP), jnp.bfloat16)
    for j in range(4):
        counts += (qn[:, j:j + 1] == iot).astype(jnp.bfloat16)
    qm1 = _dot(counts, t1a_ref[...].astype(jnp.bfloat16)) * 0.25
    t0b = jnp.tanh(_dot(t0a_ref[...] + qm1, W0_ref[...]) + b0_ref[...])
    qm2 = _dot(counts, t1b_ref[...].astype(jnp.bfloat16)) * 0.25
    t0c = jnp.tanh(_dot(t0b + qm2, W0_ref[...]) + b0_ref[...])
    qfin = jnp.tanh(_dot(t0c, lw_ref[...]) + lb_ref[...])
    tall_ref[0] = jnp.maximum(_dot(eq_ref[...], fw_ref[...]) + fb_ref[...], 0.0)
    tall_ref[1] = jnp.maximum(_dot(qfin, fw_ref[...]) + fb_ref[...], 0.0)
    tall_ref[2] = eq_ref[...]
    qs = qs_ref[...]
    esum = _dot(qs.astype(jnp.bfloat16), es_ref[...].astype(jnp.bfloat16))
    rs = jnp.sum(qs, axis=1, keepdims=True)
    tall_ref[3] = esum / jnp.maximum(rs, 1.0)


def _passC(qn, emb_q, t0a, qs_table, t1a, t1b, emb_s, W0, b0, lw, lb, fw, fb):
    return pl.pallas_call(
        _passC_body,
        grid=(GRID_Q,),
        in_specs=[
            pl.BlockSpec((BQ, 4), lambda i: (i, 0)),
            pl.BlockSpec((BQ, EMB), lambda i: (i, 0)),
            pl.BlockSpec((BQ, EMB), lambda i: (i, 0)),
            pl.BlockSpec((BQ, NUM_S), lambda i: (i, 0)),
            pl.BlockSpec((NSP, EMB), lambda i: (0, 0)),
            pl.BlockSpec((NSP, EMB), lambda i: (0, 0)),
            pl.BlockSpec((NUM_S, EMB), lambda i: (0, 0)),
            pl.BlockSpec((EMB, EMB), lambda i: (0, 0)),
            pl.BlockSpec((1, EMB), lambda i: (0, 0)),
            pl.BlockSpec((EMB, EMB), lambda i: (0, 0)),
            pl.BlockSpec((1, EMB), lambda i: (0, 0)),
            pl.BlockSpec((EMB, EMB), lambda i: (0, 0)),
            pl.BlockSpec((1, EMB), lambda i: (0, 0)),
        ],
        out_specs=pl.BlockSpec((4, BQ, EMB), lambda i: (0, i, 0)),
        out_shape=jax.ShapeDtypeStruct((4, NUM_Q, EMB), F32),
    )(qn, emb_q, t0a, qs_table, t1a, t1b, emb_s, W0, b0, lw, lb, fw, fb)


def _passR_body(xsel_ref, eqn_ref, esk_ref, resp_ref, WihAT_ref, WihBT_ref,
                WhhT_ref, bsumT_ref, qWT_ref, qbT_ref, kWT_ref, kbT_ref,
                w1_ref, w2_ref, er_ref, h0T_ref, c0T_ref, out_ref,
                hT_s, cT_s, qring, lqring):
    t = pl.program_id(0)

    @pl.when(t == 0)
    def _init():
        hT_s[...] = h0T_ref[...]
        cT_s[...] = c0T_ref[...]

    bf = jnp.bfloat16
    rWT = _dot_bt(WihBT_ref[...], er_ref[...])
    resp = resp_ref[0]
    gatesT = (_dot_bt(WihAT_ref[...].astype(bf), xsel_ref[0, 0].astype(bf))
              + _dot(WhhT_ref[...].astype(bf), hT_s[...].astype(bf))
              + bsumT_ref[...]
              + rWT[:, 0:1] + resp * (rWT[:, 1:2] - rWT[:, 0:1]))
    ig = jax.nn.sigmoid(gatesT[0:EMB])
    fg = jax.nn.sigmoid(gatesT[EMB:2 * EMB])
    gg = jnp.tanh(gatesT[2 * EMB:3 * EMB])
    og = jax.nn.sigmoid(gatesT[3 * EMB:4 * EMB])
    cT = fg * cT_s[...] + ig * gg
    hT = og * jnp.tanh(cT)
    cT_s[...] = cT
    hT_s[...] = hT

    qhT = _dot(qWT_ref[...].astype(bf), hT.astype(bf)) + qbT_ref[...]
    lq_t = jnp.sum(qhT * w1_ref[...], axis=0, keepdims=True)
    slot = lax.rem(t, RANK_K)
    qring[pl.ds(slot, 1)] = qhT[None]
    lqring[pl.ds(slot, 1)] = lq_t[None]

    kWb = kWT_ref[...].astype(bf)
    KmT0 = _dot_bt(kWb, eqn_ref[0, 0].astype(bf)) + kbT_ref[...]
    KmT1 = _dot_bt(kWb, esk_ref[0, 0].astype(bf)) + kbT_ref[...]
    w2 = w2_ref[...]
    lk0 = jnp.sum(KmT0 * w2, axis=0, keepdims=True)
    lk1 = jnp.sum(KmT1 * w2, axis=0, keepdims=True)

    ls, gs = [], []
    mx = jnp.full((1, B), -1e30, F32)
    for s_i in range(RANK_K):
        valid = jnp.logical_or(s_i <= t, t >= RANK_K)
        qrowT = qring[s_i]
        lq_s = lqring[s_i]
        for km, lk in ((KmT0, lk0), (KmT1, lk1)):
            l = jnp.where(valid, lq_s + lk, -1e30)
            g = jnp.where(
                valid,
                jax.nn.sigmoid(jnp.sum(qrowT * km, axis=0, keepdims=True)),
                0.0)
            ls.append(l)
            gs.append(g)
            mx = jnp.maximum(mx, l)
    num = jnp.zeros((1, B), F32)
    den = jnp.zeros((1, B), F32)
    for l, g in zip(ls, gs):
        e = jnp.exp(l - mx)
        num += e * g
        den += e
    out_ref[0] = num / den


def _passR(xsel, eqn, esk, respf, WihAT, WihBT, WhhT, bsumT, qWT, qbT,
           kWT, kbT, w1, w2, emb_r, h0T, c0T):
    full = lambda shape: pl.BlockSpec(shape, lambda t: tuple(0 for _ in shape))
    return pl.pallas_call(
        _passR_body,
        grid=(T,),
        in_specs=[
            pl.BlockSpec((1, 1, B, EMB), lambda t: (0, t, 0, 0)),
            pl.BlockSpec((1, 1, B, EMB), lambda t: (1, t, 0, 0)),
            pl.BlockSpec((1, 1, B, EMB), lambda t: (2, t, 0, 0)),
            pl.BlockSpec((1, 1, B), lambda t: (t, 0, 0)),
            full((4 * EMB, EMB)),
            full((4 * EMB, EMB)),
            full((4 * EMB, EMB)),
            full((4 * EMB, 1)),
            full((EMB, EMB)),
            full((EMB, 1)),
            full((EMB, EMB)),
            full((EMB, 1)),
            full((EMB, 1)),
            full((EMB, 1)),
            full((2, EMB)),
            full((EMB, B)),
            full((EMB, B)),
        ],
        out_specs=pl.BlockSpec((1, 1, B), lambda t: (t, 0, 0)),
        out_shape=jax.ShapeDtypeStruct((T, 1, B), F32),
        scratch_shapes=[
            pltpu.VMEM((EMB, B), F32),
            pltpu.VMEM((EMB, B), F32),
            pltpu.VMEM((RANK_K, EMB, B), F32),
            pltpu.VMEM((RANK_K, 1, B), F32),
        ],
    )(xsel, eqn, esk, respf, WihAT, WihBT, WhhT, bsumT, qWT, qbT, kWT, kbT,
      w1, w2, emb_r, h0T, c0T)


def kernel(question, response, mask, q_neighbors, s_neighbors, qs_table,
           emb_q, emb_s, emb_r, ft_W, ft_b, agg_W, agg_b, last_W, last_b,
           Wih, Whh, bih, bhh, q_W, q_b, k_W, k_b, w_W, w_b):
    q = question.astype(jnp.int32)
    msk = mask.astype(jnp.int32)
    qn = q_neighbors.astype(jnp.int32)
    sn = s_neighbors.astype(jnp.int32)
    W0, W1, W2 = agg_W[0], agg_W[1], agg_W[2]
    b0 = agg_b[0].reshape(1, EMB)
    b1 = agg_b[1].reshape(1, EMB)
    b2 = agg_b[2].reshape(1, EMB)
    lb = last_b.reshape(1, EMB)
    fb = ft_b.reshape(1, EMB)
    es512 = jnp.pad(emb_s, ((0, NSP - NUM_S), (0, 0)))

    tall_a, t0a = _passA(qn, emb_q, es512, W2, b2, W0, b0)

    snp = jnp.pad(sn, ((0, NSP - NUM_S), (0, 0))).T.reshape(-1)
    idx1 = jnp.concatenate([snp, snp + NUM_Q])
    g1 = _sc_gather(tall_a.reshape(2 * NUM_Q, EMB), idx1, 4096, 128)

    t1a, t1b = _passB(es512, g1.reshape(8, NSP, EMB), W1, b1)

    tall = _passC(qn, emb_q, t0a, qs_table, t1a, t1b, emb_s,
                  W0, b0, last_W, lb, ft_W, fb)

    qT = q.T
    sel = (qT[:T] + NUM_Q * msk.T[:T]).reshape(-1)
    nxt = qT[1:].reshape(-1)
    idx2 = jnp.concatenate([sel, 2 * NUM_Q + nxt, 3 * NUM_Q + nxt])
    g2 = _sc_gather(tall.reshape(4 * NUM_Q, EMB), idx2, 3 * T * B, 96)
    g2 = g2.reshape(3, T, B, EMB)

    a = float(np.sqrt(6.0 / (B + EMB)))
    kh = jax.random.split(jax.random.key(42))
    h0 = jax.random.uniform(kh[0], (B, EMB), minval=-a, maxval=a, dtype=F32)
    c0 = jax.random.uniform(kh[1], (B, EMB), minval=-a, maxval=a, dtype=F32)

    respf = response.astype(F32).T[:T].reshape(T, 1, B)
    bsumT = (bih + bhh).reshape(4 * EMB, 1)
    outp = _passR(g2, g2, g2, respf, Wih[:EMB].T, Wih[EMB:].T,
                  Whh.T, bsumT, q_W.T, q_b.reshape(EMB, 1), k_W.T,
                  k_b.reshape(EMB, 1), w_W[:EMB], w_W[EMB:], emb_r,
                  h0.T, c0.T)

    res = outp.reshape(T, B).T
    return jnp.concatenate([jnp.zeros((B, 1), F32), res], axis=1)
